# Initial kernel scaffold; baseline (speedup 1.0000x reference)
import functools

import jax
import jax.numpy as jnp
from jax import lax
from jax.experimental import pallas as pl
from jax.experimental.pallas import tpu as pltpu
from jax.experimental.pallas import tpu_sc as plsc

N = 10000
NP = 10240
E = 320000
D = 128
G = 256
EPS = 1e-5

NC, NS = 2, 16
NW = NC * NS
EPW = E // NW
RPT = NP // NS

BLK = 256
NBLK = NP // BLK

_f32 = jnp.float32
_i32 = jnp.int32

_MESH = plsc.VectorSubcoreMesh(core_axis_name="c", subcore_axis_name="s")


@functools.partial(
    pl.kernel,
    out_type=jax.ShapeDtypeStruct((NC, NP), _f32),
    mesh=_MESH,
    scratch_types=[
        pltpu.VMEM_SHARED((NP,), _f32),
        pltpu.VMEM((80,), _i32),
        pltpu.VMEM((80,), _f32),
        pltpu.VMEM((RPT,), _f32),
    ],
)
def _deg_kernel(dst_hbm, out_hbm, acc, idxv, onesv, zv):
    c = lax.axis_index("c")
    s = lax.axis_index("s")
    wid = c * NS + s

    def _zs(i, _):
        zv[pl.ds(i * 16, 16)] = jnp.zeros((16,), _f32)
        return 0
    lax.fori_loop(0, RPT // 16, _zs, 0)
    pltpu.sync_copy(zv, acc.at[pl.ds(s * RPT, RPT)])

    def _os(i, _):
        onesv[pl.ds(i * 16, 16)] = jnp.ones((16,), _f32)
        return 0
    lax.fori_loop(0, 5, _os, 0)
    plsc.subcore_barrier()

    base0 = wid * EPW

    def _chunk(j, _):
        pltpu.sync_copy(dst_hbm.at[pl.ds(base0 + j * 80, 80)], idxv)
        pltpu.sync_copy(onesv, acc.at[idxv], add=True)
        return 0
    lax.fori_loop(0, EPW // 80, _chunk, 0)
    plsc.subcore_barrier()

    pltpu.sync_copy(acc.at[pl.ds(s * RPT, RPT)], out_hbm.at[c, pl.ds(s * RPT, RPT)])


_MK = 128
_MT = EPW % _MK
_MCH = EPW // _MK


@functools.partial(
    pl.kernel,
    out_type=jax.ShapeDtypeStruct((NC, NP, D), _f32),
    mesh=_MESH,
    scratch_types=[
        pltpu.VMEM_SHARED((NP, D), _f32),
        pltpu.VMEM((_MK, D), _f32),
        pltpu.VMEM((_MK,), _i32),
        pltpu.VMEM((_MK,), _i32),
        pltpu.VMEM((_MT, D), _f32),
        pltpu.VMEM((_MT,), _i32),
        pltpu.VMEM((_MT,), _i32),
        pltpu.VMEM((128, D), _f32),
    ],
)
def _msg_kernel(hs_hbm, src_hbm, dst_hbm, out_hbm, acc, rows, srcv, dstv,
                rows_t, srcv_t, dstv_t, zbuf):
    c = lax.axis_index("c")
    s = lax.axis_index("s")
    wid = c * NS + s

    def _zrow(i, _):
        for cc in range(8):
            zbuf[i, pl.ds(cc * 16, 16)] = jnp.zeros((16,), _f32)
        return 0
    lax.fori_loop(0, 128, _zrow, 0)
    for b in range(RPT // 128):
        pltpu.sync_copy(zbuf, acc.at[pl.ds(s * RPT + b * 128, 128)])
    plsc.subcore_barrier()

    base0 = wid * EPW

    def _chunk(j, _):
        base = base0 + j * _MK
        pltpu.sync_copy(src_hbm.at[pl.ds(base, _MK)], srcv)
        pltpu.sync_copy(dst_hbm.at[pl.ds(base, _MK)], dstv)
        pltpu.sync_copy(hs_hbm.at[srcv], rows)
        pltpu.sync_copy(rows, acc.at[dstv], add=True)
        return 0
    lax.fori_loop(0, _MCH, _chunk, 0)

    base = base0 + _MCH * _MK
    pltpu.sync_copy(src_hbm.at[pl.ds(base, _MT)], srcv_t)
    pltpu.sync_copy(dst_hbm.at[pl.ds(base, _MT)], dstv_t)
    pltpu.sync_copy(hs_hbm.at[srcv_t], rows_t)
    pltpu.sync_copy(rows_t, acc.at[dstv_t], add=True)
    plsc.subcore_barrier()

    pltpu.sync_copy(acc.at[pl.ds(s * RPT, RPT)],
                    out_hbm.at[c, pl.ds(s * RPT, RPT)])


_SC_C = 64


@functools.partial(
    pl.kernel,
    out_type=jax.ShapeDtypeStruct((G, D), _f32),
    mesh=_MESH,
    scratch_types=[
        pltpu.VMEM((8 * D,), _f32),
        pltpu.VMEM((_SC_C, D), _f32),
        pltpu.VMEM((80,), _i32),
        pltpu.VMEM((256,), _f32),
        pltpu.VMEM((16,), _i32),
        pltpu.VMEM((8, D), _f32),
    ],
)
def _segmax_kernel(r_hbm, ib_hbm, starts_hbm, st_hbm, out_hbm,
                   accv, rowbuf, ibv, stv, sv, obuf):
    c = lax.axis_index("c")
    s = lax.axis_index("s")
    wid = c * NS + s
    g0 = wid * 8

    pltpu.sync_copy(starts_hbm.at[pl.ds(g0, 16)], sv)
    pltpu.sync_copy(st_hbm, stv)
    start = sv[0]
    end = sv[8]

    def _ini(i, _):
        accv[pl.ds(i * 16, 16)] = jnp.full((16,), -jnp.inf, _f32)
        return 0
    lax.fori_loop(0, 8 * D // 16, _ini, 0)

    nch = lax.div(end - start + (_SC_C - 1), _SC_C)

    def _chunk(t, _):
        base = start + t * _SC_C
        nrows = jnp.minimum(_SC_C, end - base)
        pltpu.sync_copy(r_hbm.at[pl.ds(base, _SC_C)], rowbuf)
        ib_base = lax.div(base, 8) * 8
        shift = base - ib_base
        pltpu.sync_copy(ib_hbm.at[pl.ds(ib_base, 80)], ibv)

        def _row(i, _):
            ib = ibv[shift + i]
            off = (ib - g0) * D
            for cc in range(8):
                a = accv[pl.ds(off + cc * 16, 16)]
                v = rowbuf[i, pl.ds(cc * 16, 16)]
                accv[pl.ds(off + cc * 16, 16)] = jnp.maximum(a, v)
            return 0
        lax.fori_loop(0, nrows, _row, 0)
        return 0
    lax.fori_loop(0, nch, _chunk, 0)

    for k in range(8):
        for cc in range(8):
            s2 = stv[pl.ds(cc * 16, 16)]
            t2 = stv[pl.ds(D + cc * 16, 16)]
            obuf[k, pl.ds(cc * 16, 16)] = accv[pl.ds(k * D + cc * 16, 16)] * s2 + t2
    pltpu.sync_copy(obuf, out_hbm.at[pl.ds(g0, 8)])


def _k1_body(x_ref, deg_ref, w_ref, hs_ref, dinv_ref):
    dinv = lax.rsqrt(deg_ref[...])
    h = jnp.dot(x_ref[...], w_ref[...], preferred_element_type=_f32)
    hs_ref[...] = h * dinv
    dinv_ref[...] = dinv


def _k1(x_pad, degcol, W):
    return pl.pallas_call(
        _k1_body,
        grid=(NBLK,),
        in_specs=[
            pl.BlockSpec((BLK, D), lambda i: (i, 0)),
            pl.BlockSpec((BLK, 1), lambda i: (i, 0)),
            pl.BlockSpec((D, D), lambda i: (0, 0)),
        ],
        out_specs=[
            pl.BlockSpec((BLK, D), lambda i: (i, 0)),
            pl.BlockSpec((BLK, 1), lambda i: (i, 0)),
        ],
        out_shape=[
            jax.ShapeDtypeStruct((NP, D), _f32),
            jax.ShapeDtypeStruct((NP, 1), _f32),
        ],
    )(x_pad, degcol, W)


def _k2_body(p0_ref, p1_ref, hs_ref, dinv_ref, b_ref, g_ref, be_ref,
             r_ref, st_ref, ssum, ssq):
    i = pl.program_id(0)
    v = dinv_ref[...] * (p0_ref[...] + p1_ref[...] + hs_ref[...]) + b_ref[...]
    r = jnp.maximum(v, 0.0)
    row = i * BLK + lax.broadcasted_iota(_i32, (BLK, 1), 0)
    r = jnp.where(row < N, r, 0.0)
    r_ref[...] = r
    cs = jnp.sum(r, axis=0, keepdims=True)
    cq = jnp.sum(r * r, axis=0, keepdims=True)

    @pl.when(i == 0)
    def _():
        ssum[...] = cs
        ssq[...] = cq

    @pl.when(i > 0)
    def _():
        ssum[...] += cs
        ssq[...] += cq

    @pl.when(i == NBLK - 1)
    def _():
        mean = ssum[...] / N
        var = ssq[...] / N - mean * mean
        sc = g_ref[...] * lax.rsqrt(var + EPS)
        st_ref[...] = jnp.concatenate([sc, be_ref[...] - mean * sc], axis=0)


def _k2(p0, p1, hs, dinvcol, b, g, be):
    return pl.pallas_call(
        _k2_body,
        grid=(NBLK,),
        in_specs=[
            pl.BlockSpec((BLK, D), lambda i: (i, 0)),
            pl.BlockSpec((BLK, D), lambda i: (i, 0)),
            pl.BlockSpec((BLK, D), lambda i: (i, 0)),
            pl.BlockSpec((BLK, 1), lambda i: (i, 0)),
            pl.BlockSpec((1, D), lambda i: (0, 0)),
            pl.BlockSpec((1, D), lambda i: (0, 0)),
            pl.BlockSpec((1, D), lambda i: (0, 0)),
        ],
        out_specs=[
            pl.BlockSpec((BLK, D), lambda i: (i, 0)),
            pl.BlockSpec((2, D), lambda i: (0, 0)),
        ],
        out_shape=[
            jax.ShapeDtypeStruct((NP, D), _f32),
            jax.ShapeDtypeStruct((2, D), _f32),
        ],
        scratch_shapes=[
            pltpu.VMEM((1, D), _f32),
            pltpu.VMEM((1, D), _f32),
        ],
    )(p0, p1, hs, dinvcol, b, g, be)


def _k3_body(r_ref, st_ref, w_ref, dinv_ref, hs2_ref):
    sc = st_ref[0:1, :]
    sh = st_ref[1:2, :]
    x2 = r_ref[...] * sc + sh
    h2 = jnp.dot(x2, w_ref[...], preferred_element_type=_f32)
    hs2_ref[...] = h2 * dinv_ref[...]


def _k3(r, st, W, dinvcol):
    return pl.pallas_call(
        _k3_body,
        grid=(NBLK,),
        in_specs=[
            pl.BlockSpec((BLK, D), lambda i: (i, 0)),
            pl.BlockSpec((2, D), lambda i: (0, 0)),
            pl.BlockSpec((D, D), lambda i: (0, 0)),
            pl.BlockSpec((BLK, 1), lambda i: (i, 0)),
        ],
        out_specs=pl.BlockSpec((BLK, D), lambda i: (i, 0)),
        out_shape=jax.ShapeDtypeStruct((NP, D), _f32),
    )(r, st, W, dinvcol)


def kernel(input_feature, input_adj, ibatch, W1, b1, g1, be1, W2, b2, g2, be2):
    src = input_adj[0]
    dst = input_adj[1]

    degp = _deg_kernel(dst)
    degcol = (degp[0] + degp[1] + 1.0)[:, None]

    x_pad = jnp.pad(input_feature, ((0, NP - N), (0, 0)))
    hs1, dinvcol = _k1(x_pad, degcol, W1)

    parts1 = _msg_kernel(hs1, src, dst)
    r1, st1 = _k2(parts1[0], parts1[1], hs1, dinvcol,
                  b1[None, :], g1[None, :], be1[None, :])

    hs2 = _k3(r1, st1, W2, dinvcol)
    parts2 = _msg_kernel(hs2, src, dst)
    r2, st2 = _k2(parts2[0], parts2[1], hs2, dinvcol,
                  b2[None, :], g2[None, :], be2[None, :])

    starts = jnp.searchsorted(ibatch, jnp.arange(G + 1, dtype=_i32)).astype(_i32)
    starts = jnp.concatenate([starts, jnp.full((7,), N, _i32)])
    ib_pad = jnp.pad(ibatch, (0, NP - N), constant_values=G - 1)

    out = _segmax_kernel(r2, ib_pad, starts, st2.reshape(2 * D))
    return out


# speedup vs baseline: 13.6243x; 13.6243x over previous
import functools

import jax
import jax.numpy as jnp
from jax import lax
from jax.experimental import pallas as pl
from jax.experimental.pallas import tpu as pltpu
from jax.experimental.pallas import tpu_sc as plsc

N = 10000
NP = 10240
E = 320000
D = 128
G = 256
EPS = 1e-5

NC, NS = 2, 16
NW = NC * NS
EPW = E // NW
RPT = NP // NS

BLK = 256
NBLK = NP // BLK

_f32 = jnp.float32
_i32 = jnp.int32

@functools.cache
def _mesh():
    return plsc.VectorSubcoreMesh(core_axis_name="c", subcore_axis_name="s",
                                  num_cores=NC, num_subcores=NS)


@functools.cache
def _deg_kernel_fn():
    return pl.kernel(
        _deg_body,
        out_type=jax.ShapeDtypeStruct((NC, NP), _f32),
        mesh=_mesh(),
        scratch_types=[
            pltpu.VMEM_SHARED((NP,), _f32),
            pltpu.VMEM((80,), _i32),
            pltpu.VMEM((80,), _f32),
            pltpu.VMEM((RPT,), _f32),
        ],
    )


def _deg_body(dst_hbm, out_hbm, acc, idxv, onesv, zv):
    c = lax.axis_index("c")
    s = lax.axis_index("s")
    wid = c * NS + s

    def _zs(i, _):
        zv[pl.ds(i * 16, 16)] = jnp.zeros((16,), _f32)
        return 0
    lax.fori_loop(0, RPT // 16, _zs, 0)
    pltpu.sync_copy(zv, acc.at[pl.ds(s * RPT, RPT)])

    def _os(i, _):
        onesv[pl.ds(i * 16, 16)] = jnp.ones((16,), _f32)
        return 0
    lax.fori_loop(0, 5, _os, 0)
    plsc.subcore_barrier()

    base0 = wid * EPW

    def _chunk(j, _):
        pltpu.sync_copy(dst_hbm.at[pl.ds(base0 + j * 80, 80)], idxv)
        pltpu.sync_copy(onesv, acc.at[idxv], add=True)
        return 0
    lax.fori_loop(0, EPW // 80, _chunk, 0)
    plsc.subcore_barrier()

    pltpu.sync_copy(acc.at[pl.ds(s * RPT, RPT)], out_hbm.at[c, pl.ds(s * RPT, RPT)])


_MK = 128
_MT = EPW % _MK
_MCH = EPW // _MK


@functools.cache
def _msg_kernel_fn():
    return pl.kernel(
        _msg_body,
        out_type=jax.ShapeDtypeStruct((NC, NP, D), _f32),
        mesh=_mesh(),
        scratch_types=[
            pltpu.VMEM_SHARED((NP, D), _f32),
            pltpu.VMEM((_MK, D), _f32),
            pltpu.VMEM((_MK,), _i32),
            pltpu.VMEM((_MK,), _i32),
            pltpu.VMEM((_MT, D), _f32),
            pltpu.VMEM((_MT,), _i32),
            pltpu.VMEM((_MT,), _i32),
            pltpu.VMEM((128, D), _f32),
        ],
    )


def _msg_body(hs_hbm, src_hbm, dst_hbm, out_hbm, acc, rows, srcv, dstv,
              rows_t, srcv_t, dstv_t, zbuf):
    c = lax.axis_index("c")
    s = lax.axis_index("s")
    wid = c * NS + s

    def _zrow(i, _):
        for cc in range(8):
            zbuf[i, pl.ds(cc * 16, 16)] = jnp.zeros((16,), _f32)
        return 0
    lax.fori_loop(0, 128, _zrow, 0)
    for b in range(RPT // 128):
        pltpu.sync_copy(zbuf, acc.at[pl.ds(s * RPT + b * 128, 128)])
    plsc.subcore_barrier()

    base0 = wid * EPW

    def _chunk(j, _):
        base = base0 + j * _MK
        pltpu.sync_copy(src_hbm.at[pl.ds(base, _MK)], srcv)
        pltpu.sync_copy(dst_hbm.at[pl.ds(base, _MK)], dstv)
        pltpu.sync_copy(hs_hbm.at[srcv], rows)
        pltpu.sync_copy(rows, acc.at[dstv], add=True)
        return 0
    lax.fori_loop(0, _MCH, _chunk, 0)

    base = base0 + _MCH * _MK
    pltpu.sync_copy(src_hbm.at[pl.ds(base, _MT)], srcv_t)
    pltpu.sync_copy(dst_hbm.at[pl.ds(base, _MT)], dstv_t)
    pltpu.sync_copy(hs_hbm.at[srcv_t], rows_t)
    pltpu.sync_copy(rows_t, acc.at[dstv_t], add=True)
    plsc.subcore_barrier()

    pltpu.sync_copy(acc.at[pl.ds(s * RPT, RPT)],
                    out_hbm.at[c, pl.ds(s * RPT, RPT)])


_SC_C = 64


@functools.cache
def _segmax_kernel_fn():
    return pl.kernel(
        _segmax_body,
        out_type=jax.ShapeDtypeStruct((G, D), _f32),
        mesh=_mesh(),
        scratch_types=[
            pltpu.VMEM((8 * D,), _f32),
            pltpu.VMEM((_SC_C + 8, D), _f32),
            pltpu.VMEM((96,), _i32),
            pltpu.VMEM((256,), _f32),
            pltpu.VMEM((16,), _i32),
            pltpu.VMEM((8, D), _f32),
        ],
    )


def _segmax_body(r_hbm, ib_hbm, starts_hbm, st_hbm, out_hbm,
                 accv, rowbuf, ibv, stv, sv, obuf):
    c = lax.axis_index("c")
    s = lax.axis_index("s")
    wid = c * NS + s
    g0 = wid * 8

    pltpu.sync_copy(starts_hbm.at[pl.ds(g0, 16)], sv)
    pltpu.sync_copy(st_hbm, stv)
    svv = sv[pl.ds(0, 16)]
    start = svv[0]
    end = svv[8]

    def _ini(i, _):
        accv[pl.ds(i * 16, 16)] = jnp.full((16,), -jnp.inf, _f32)
        return 0
    lax.fori_loop(0, 8 * D // 16, _ini, 0)

    nch = lax.div(end - start + (_SC_C - 1), _SC_C)

    def _chunk(t, _):
        base = start + t * _SC_C
        nrows = jnp.minimum(_SC_C, end - base)
        ab = lax.div(base, 8) * 8
        sh = base - ab
        pltpu.sync_copy(r_hbm.at[pl.ds(ab, _SC_C + 8)], rowbuf)
        pltpu.sync_copy(ib_hbm.at[pl.ds(ab, 80)], ibv.at[pl.ds(0, 80)])

        def _row(i, _):
            ib = ibv[pl.ds(sh + i, 16)][0]
            off = (ib - g0) * D
            for cc in range(8):
                a = accv[pl.ds(off + cc * 16, 16)]
                v = rowbuf[sh + i, pl.ds(cc * 16, 16)]
                accv[pl.ds(off + cc * 16, 16)] = jnp.maximum(a, v)
            return 0
        lax.fori_loop(0, nrows, _row, 0)
        return 0
    lax.fori_loop(0, nch, _chunk, 0)

    for k in range(8):
        for cc in range(8):
            s2 = stv[pl.ds(cc * 16, 16)]
            t2 = stv[pl.ds(D + cc * 16, 16)]
            obuf[k, pl.ds(cc * 16, 16)] = accv[pl.ds(k * D + cc * 16, 16)] * s2 + t2
    pltpu.sync_copy(obuf, out_hbm.at[pl.ds(g0, 8)])


def _k1_body(x_ref, deg_ref, w_ref, hs_ref, dinv_ref):
    dinv = lax.rsqrt(deg_ref[...])
    h = jnp.dot(x_ref[...], w_ref[...], preferred_element_type=_f32)
    hs_ref[...] = h * dinv
    dinv_ref[...] = dinv


def _k1(x_pad, degcol, W):
    return pl.pallas_call(
        _k1_body,
        grid=(NBLK,),
        in_specs=[
            pl.BlockSpec((BLK, D), lambda i: (i, 0)),
            pl.BlockSpec((BLK, 1), lambda i: (i, 0)),
            pl.BlockSpec((D, D), lambda i: (0, 0)),
        ],
        out_specs=[
            pl.BlockSpec((BLK, D), lambda i: (i, 0)),
            pl.BlockSpec((BLK, 1), lambda i: (i, 0)),
        ],
        out_shape=[
            jax.ShapeDtypeStruct((NP, D), _f32),
            jax.ShapeDtypeStruct((NP, 1), _f32),
        ],
    )(x_pad, degcol, W)


def _k2_body(p0_ref, p1_ref, hs_ref, dinv_ref, b_ref, g_ref, be_ref,
             r_ref, st_ref, ssum, ssq):
    i = pl.program_id(0)
    v = dinv_ref[...] * (p0_ref[...] + p1_ref[...] + hs_ref[...]) + b_ref[...]
    r = jnp.maximum(v, 0.0)
    row = i * BLK + lax.broadcasted_iota(_i32, (BLK, 1), 0)
    r = jnp.where(row < N, r, 0.0)
    r_ref[...] = r
    cs = jnp.sum(r, axis=0, keepdims=True)
    cq = jnp.sum(r * r, axis=0, keepdims=True)

    @pl.when(i == 0)
    def _():
        ssum[...] = cs
        ssq[...] = cq

    @pl.when(i > 0)
    def _():
        ssum[...] += cs
        ssq[...] += cq

    @pl.when(i == NBLK - 1)
    def _():
        mean = ssum[...] / N
        var = ssq[...] / N - mean * mean
        sc = g_ref[...] * lax.rsqrt(var + EPS)
        st_ref[...] = jnp.concatenate([sc, be_ref[...] - mean * sc], axis=0)


def _k2(p0, p1, hs, dinvcol, b, g, be):
    return pl.pallas_call(
        _k2_body,
        grid=(NBLK,),
        in_specs=[
            pl.BlockSpec((BLK, D), lambda i: (i, 0)),
            pl.BlockSpec((BLK, D), lambda i: (i, 0)),
            pl.BlockSpec((BLK, D), lambda i: (i, 0)),
            pl.BlockSpec((BLK, 1), lambda i: (i, 0)),
            pl.BlockSpec((1, D), lambda i: (0, 0)),
            pl.BlockSpec((1, D), lambda i: (0, 0)),
            pl.BlockSpec((1, D), lambda i: (0, 0)),
        ],
        out_specs=[
            pl.BlockSpec((BLK, D), lambda i: (i, 0)),
            pl.BlockSpec((2, D), lambda i: (0, 0)),
        ],
        out_shape=[
            jax.ShapeDtypeStruct((NP, D), _f32),
            jax.ShapeDtypeStruct((2, D), _f32),
        ],
        scratch_shapes=[
            pltpu.VMEM((1, D), _f32),
            pltpu.VMEM((1, D), _f32),
        ],
    )(p0, p1, hs, dinvcol, b, g, be)


def _k3_body(r_ref, st_ref, w_ref, dinv_ref, hs2_ref):
    sc = st_ref[0:1, :]
    sh = st_ref[1:2, :]
    x2 = r_ref[...] * sc + sh
    h2 = jnp.dot(x2, w_ref[...], preferred_element_type=_f32)
    hs2_ref[...] = h2 * dinv_ref[...]


def _k3(r, st, W, dinvcol):
    return pl.pallas_call(
        _k3_body,
        grid=(NBLK,),
        in_specs=[
            pl.BlockSpec((BLK, D), lambda i: (i, 0)),
            pl.BlockSpec((2, D), lambda i: (0, 0)),
            pl.BlockSpec((D, D), lambda i: (0, 0)),
            pl.BlockSpec((BLK, 1), lambda i: (i, 0)),
        ],
        out_specs=pl.BlockSpec((BLK, D), lambda i: (i, 0)),
        out_shape=jax.ShapeDtypeStruct((NP, D), _f32),
    )(r, st, W, dinvcol)


def kernel(input_feature, input_adj, ibatch, W1, b1, g1, be1, W2, b2, g2, be2):
    src = input_adj[0]
    dst = input_adj[1]

    degp = _deg_kernel_fn()(dst)
    degcol = (degp[0] + degp[1] + 1.0)[:, None]

    x_pad = jnp.pad(input_feature, ((0, NP - N), (0, 0)))
    hs1, dinvcol = _k1(x_pad, degcol, W1)

    parts1 = _msg_kernel_fn()(hs1, src, dst)
    r1, st1 = _k2(parts1[0], parts1[1], hs1, dinvcol,
                  b1[None, :], g1[None, :], be1[None, :])

    hs2 = _k3(r1, st1, W2, dinvcol)
    parts2 = _msg_kernel_fn()(hs2, src, dst)
    r2, st2 = _k2(parts2[0], parts2[1], hs2, dinvcol,
                  b2[None, :], g2[None, :], be2[None, :])

    starts = jnp.searchsorted(ibatch, jnp.arange(G + 1, dtype=_i32)).astype(_i32)
    starts = jnp.concatenate([starts, jnp.full((7,), N, _i32)])
    ib_pad = jnp.pad(ibatch, (0, NP - N), constant_values=G - 1)

    out = _segmax_kernel_fn()(r2, ib_pad, starts, st2.reshape(2 * D))
    return out


# speedup vs baseline: 24.4887x; 1.7974x over previous
import functools

import jax
import jax.numpy as jnp
from jax import lax
from jax.experimental import pallas as pl
from jax.experimental.pallas import tpu as pltpu
from jax.experimental.pallas import tpu_sc as plsc

N = 10000
NP = 10240
E = 320000
D = 128
G = 256
EPS = 1e-5

NC, NS = 2, 16
NW = NC * NS
EPW = E // NW
RPT = NP // NS

BLK = 256
NBLK = NP // BLK

_f32 = jnp.float32
_i32 = jnp.int32

@functools.cache
def _mesh():
    return plsc.VectorSubcoreMesh(core_axis_name="c", subcore_axis_name="s",
                                  num_cores=NC, num_subcores=NS)


ER = E // 128
_MK = 128


def _worker_rows(w):
    off = jnp.where(w < 24, 80 * w, 1920 + 72 * (w - 24))
    n = jnp.where(w < 24, 80, jnp.where(w == 31, 76, 72))
    return off, n


def _load_idx_rows(src2d_hbm, buf, w, off):
    pltpu.sync_copy(src2d_hbm.at[pl.ds(off, 72)], buf.at[pl.ds(0, 72)])

    @pl.when(w < 24)
    def _():
        pltpu.sync_copy(src2d_hbm.at[pl.ds(off + 72, 8)], buf.at[pl.ds(72, 8)])

    @pl.when(w == 31)
    def _():
        pltpu.sync_copy(src2d_hbm.at[pl.ds(off + 72, 4)], buf.at[pl.ds(72, 4)])


@functools.cache
def _deg_kernel_fn():
    return pl.kernel(
        _deg_body,
        out_type=jax.ShapeDtypeStruct((NC, NP), _f32),
        mesh=_mesh(),
        scratch_types=[
            pltpu.VMEM_SHARED((NP,), _f32),
            pltpu.VMEM((80, _MK), _i32),
            pltpu.VMEM((_MK,), _f32),
            pltpu.VMEM((RPT,), _f32),
            pltpu.SemaphoreType.DMA,
        ],
    )


def _deg_body(dst2d_hbm, out_hbm, acc, idxv, onesv, zv, sem):
    c = lax.axis_index("c")
    s = lax.axis_index("s")
    wid = c * NS + s
    off, n = _worker_rows(wid)

    def _zs(i, _):
        zv[pl.ds(i * 16, 16)] = jnp.zeros((16,), _f32)
        return 0
    lax.fori_loop(0, RPT // 16, _zs, 0)
    pltpu.sync_copy(zv, acc.at[pl.ds(s * RPT, RPT)])

    def _os(i, _):
        onesv[pl.ds(i * 16, 16)] = jnp.ones((16,), _f32)
        return 0
    lax.fori_loop(0, _MK // 16, _os, 0)
    _load_idx_rows(dst2d_hbm, idxv, wid, off)
    plsc.subcore_barrier()

    def _fire(j, _):
        pltpu.async_copy(onesv, acc.at[idxv.at[j]], sem, add=True)
        return 0
    lax.fori_loop(0, n, _fire, 0)

    def _drain(j, _):
        pltpu.make_async_copy(dst2d_hbm.at[pl.ds(0, 1)],
                              idxv.at[pl.ds(0, 1)], sem).wait()
        return 0
    lax.fori_loop(0, n, _drain, 0)
    plsc.subcore_barrier()

    pltpu.sync_copy(acc.at[pl.ds(s * RPT, RPT)], out_hbm.at[c, pl.ds(s * RPT, RPT)])


_NBUF = 2


@functools.cache
def _msg_kernel_fn():
    return pl.kernel(
        _msg_body,
        out_type=jax.ShapeDtypeStruct((NC, NP, D), _f32),
        mesh=_mesh(),
        scratch_types=[
            pltpu.VMEM_SHARED((NP, D), _f32),
            [pltpu.VMEM((_MK, D), _f32)] * _NBUF,
            [pltpu.VMEM((_MK,), _i32)] * _NBUF,
            pltpu.VMEM((80, _MK), _i32),
            pltpu.VMEM((16, D), _f32),
            [pltpu.SemaphoreType.DMA] * _NBUF,
            [pltpu.SemaphoreType.DMA] * _NBUF,
        ],
    )


def _msg_body(hs_hbm, src_hbm, dst2d_hbm, out_hbm, acc, rows, srcb, dstv,
              zbuf, sem_g, sem_s):
    c = lax.axis_index("c")
    s = lax.axis_index("s")
    wid = c * NS + s
    off, n = _worker_rows(wid)

    def _zrow(i, _):
        for cc in range(8):
            zbuf[i, pl.ds(cc * 16, 16)] = jnp.zeros((16,), _f32)
        return 0
    lax.fori_loop(0, 16, _zrow, 0)
    for b in range(RPT // 16):
        pltpu.sync_copy(zbuf, acc.at[pl.ds(s * RPT + b * 16, 16)])
    _load_idx_rows(dst2d_hbm, dstv, wid, off)
    plsc.subcore_barrier()

    ebase = off * _MK

    for i in range(_NBUF):
        pltpu.sync_copy(src_hbm.at[pl.ds(ebase + i * _MK, _MK)], srcb[i])
        pltpu.async_copy(hs_hbm.at[srcb[i]], rows[i], sem_g[i])

    def _group(cg, _):
        for i in range(_NBUF):
            k = cg * _NBUF + i

            @pl.when(k < n)
            def _():
                pltpu.make_async_copy(hs_hbm.at[pl.ds(0, _MK)], rows[i],
                                      sem_g[i]).wait()
                desc = pltpu.async_copy(rows[i], acc.at[dstv.at[k]],
                                        sem_s[i], add=True)

                @pl.when(k + _NBUF < n)
                def _():
                    pltpu.sync_copy(
                        src_hbm.at[pl.ds(ebase + (k + _NBUF) * _MK, _MK)],
                        srcb[i])
                    desc.wait()
                    pltpu.async_copy(hs_hbm.at[srcb[i]], rows[i], sem_g[i])
        return 0
    lax.fori_loop(0, 40, _group, 0)

    for i in range(_NBUF):
        pltpu.make_async_copy(hs_hbm.at[pl.ds(0, _MK)], rows[i],
                              sem_s[i]).wait()
    plsc.subcore_barrier()

    pltpu.sync_copy(acc.at[pl.ds(s * RPT, RPT)],
                    out_hbm.at[c, pl.ds(s * RPT, RPT)])


_SC_C = 64


@functools.cache
def _segmax_kernel_fn():
    return pl.kernel(
        _segmax_body,
        out_type=jax.ShapeDtypeStruct((G, D), _f32),
        mesh=_mesh(),
        scratch_types=[
            pltpu.VMEM((8 * D,), _f32),
            pltpu.VMEM((_SC_C + 8, D), _f32),
            pltpu.VMEM((96,), _i32),
            pltpu.VMEM((256,), _f32),
            pltpu.VMEM((16,), _i32),
            pltpu.VMEM((8, D), _f32),
        ],
    )


def _segmax_body(r_hbm, ib_hbm, starts_hbm, st_hbm, out_hbm,
                 accv, rowbuf, ibv, stv, sv, obuf):
    c = lax.axis_index("c")
    s = lax.axis_index("s")
    wid = c * NS + s
    g0 = wid * 8

    pltpu.sync_copy(starts_hbm.at[pl.ds(g0, 16)], sv)
    pltpu.sync_copy(st_hbm, stv)
    svv = sv[pl.ds(0, 16)]
    start = svv[0]
    end = svv[8]

    def _ini(i, _):
        accv[pl.ds(i * 16, 16)] = jnp.full((16,), -jnp.inf, _f32)
        return 0
    lax.fori_loop(0, 8 * D // 16, _ini, 0)

    nch = lax.div(end - start + (_SC_C - 1), _SC_C)

    def _chunk(t, _):
        base = start + t * _SC_C
        nrows = jnp.minimum(_SC_C, end - base)
        ab = lax.div(base, 8) * 8
        sh = base - ab
        pltpu.sync_copy(r_hbm.at[pl.ds(ab, _SC_C + 8)], rowbuf)
        pltpu.sync_copy(ib_hbm.at[pl.ds(ab, 80)], ibv.at[pl.ds(0, 80)])

        def _row(i, _):
            ib = ibv[pl.ds(sh + i, 16)][0]
            off = (ib - g0) * D
            for cc in range(8):
                a = accv[pl.ds(off + cc * 16, 16)]
                v = rowbuf[sh + i, pl.ds(cc * 16, 16)]
                accv[pl.ds(off + cc * 16, 16)] = jnp.maximum(a, v)
            return 0
        lax.fori_loop(0, nrows, _row, 0)
        return 0
    lax.fori_loop(0, nch, _chunk, 0)

    for k in range(8):
        for cc in range(8):
            s2 = stv[pl.ds(cc * 16, 16)]
            t2 = stv[pl.ds(D + cc * 16, 16)]
            obuf[k, pl.ds(cc * 16, 16)] = accv[pl.ds(k * D + cc * 16, 16)] * s2 + t2
    pltpu.sync_copy(obuf, out_hbm.at[pl.ds(g0, 8)])


def _k1_body(x_ref, deg_ref, w_ref, hs_ref, dinv_ref):
    dinv = lax.rsqrt(deg_ref[...])
    h = jnp.dot(x_ref[...], w_ref[...], preferred_element_type=_f32)
    hs_ref[...] = h * dinv
    dinv_ref[...] = dinv


def _k1(x_pad, degcol, W):
    return pl.pallas_call(
        _k1_body,
        grid=(NBLK,),
        in_specs=[
            pl.BlockSpec((BLK, D), lambda i: (i, 0)),
            pl.BlockSpec((BLK, 1), lambda i: (i, 0)),
            pl.BlockSpec((D, D), lambda i: (0, 0)),
        ],
        out_specs=[
            pl.BlockSpec((BLK, D), lambda i: (i, 0)),
            pl.BlockSpec((BLK, 1), lambda i: (i, 0)),
        ],
        out_shape=[
            jax.ShapeDtypeStruct((NP, D), _f32),
            jax.ShapeDtypeStruct((NP, 1), _f32),
        ],
    )(x_pad, degcol, W)


def _k2_body(p0_ref, p1_ref, hs_ref, dinv_ref, b_ref, g_ref, be_ref,
             r_ref, st_ref, ssum, ssq):
    i = pl.program_id(0)
    v = dinv_ref[...] * (p0_ref[...] + p1_ref[...] + hs_ref[...]) + b_ref[...]
    r = jnp.maximum(v, 0.0)
    row = i * BLK + lax.broadcasted_iota(_i32, (BLK, 1), 0)
    r = jnp.where(row < N, r, 0.0)
    r_ref[...] = r
    cs = jnp.sum(r, axis=0, keepdims=True)
    cq = jnp.sum(r * r, axis=0, keepdims=True)

    @pl.when(i == 0)
    def _():
        ssum[...] = cs
        ssq[...] = cq

    @pl.when(i > 0)
    def _():
        ssum[...] += cs
        ssq[...] += cq

    @pl.when(i == NBLK - 1)
    def _():
        mean = ssum[...] / N
        var = ssq[...] / N - mean * mean
        sc = g_ref[...] * lax.rsqrt(var + EPS)
        st_ref[...] = jnp.concatenate([sc, be_ref[...] - mean * sc], axis=0)


def _k2(p0, p1, hs, dinvcol, b, g, be):
    return pl.pallas_call(
        _k2_body,
        grid=(NBLK,),
        in_specs=[
            pl.BlockSpec((BLK, D), lambda i: (i, 0)),
            pl.BlockSpec((BLK, D), lambda i: (i, 0)),
            pl.BlockSpec((BLK, D), lambda i: (i, 0)),
            pl.BlockSpec((BLK, 1), lambda i: (i, 0)),
            pl.BlockSpec((1, D), lambda i: (0, 0)),
            pl.BlockSpec((1, D), lambda i: (0, 0)),
            pl.BlockSpec((1, D), lambda i: (0, 0)),
        ],
        out_specs=[
            pl.BlockSpec((BLK, D), lambda i: (i, 0)),
            pl.BlockSpec((2, D), lambda i: (0, 0)),
        ],
        out_shape=[
            jax.ShapeDtypeStruct((NP, D), _f32),
            jax.ShapeDtypeStruct((2, D), _f32),
        ],
        scratch_shapes=[
            pltpu.VMEM((1, D), _f32),
            pltpu.VMEM((1, D), _f32),
        ],
    )(p0, p1, hs, dinvcol, b, g, be)


def _k3_body(r_ref, st_ref, w_ref, dinv_ref, hs2_ref):
    sc = st_ref[0:1, :]
    sh = st_ref[1:2, :]
    x2 = r_ref[...] * sc + sh
    h2 = jnp.dot(x2, w_ref[...], preferred_element_type=_f32)
    hs2_ref[...] = h2 * dinv_ref[...]


def _k3(r, st, W, dinvcol):
    return pl.pallas_call(
        _k3_body,
        grid=(NBLK,),
        in_specs=[
            pl.BlockSpec((BLK, D), lambda i: (i, 0)),
            pl.BlockSpec((2, D), lambda i: (0, 0)),
            pl.BlockSpec((D, D), lambda i: (0, 0)),
            pl.BlockSpec((BLK, 1), lambda i: (i, 0)),
        ],
        out_specs=pl.BlockSpec((BLK, D), lambda i: (i, 0)),
        out_shape=jax.ShapeDtypeStruct((NP, D), _f32),
    )(r, st, W, dinvcol)


def kernel(input_feature, input_adj, ibatch, W1, b1, g1, be1, W2, b2, g2, be2):
    src1d = input_adj[0]
    dst2d = input_adj[1].reshape(ER, 128)

    degp = _deg_kernel_fn()(dst2d)
    degcol = (degp[0] + degp[1] + 1.0)[:, None]

    x_pad = jnp.pad(input_feature, ((0, NP - N), (0, 0)))
    hs1, dinvcol = _k1(x_pad, degcol, W1)

    parts1 = _msg_kernel_fn()(hs1, src1d, dst2d)
    r1, st1 = _k2(parts1[0], parts1[1], hs1, dinvcol,
                  b1[None, :], g1[None, :], be1[None, :])

    hs2 = _k3(r1, st1, W2, dinvcol)
    parts2 = _msg_kernel_fn()(hs2, src1d, dst2d)
    r2, st2 = _k2(parts2[0], parts2[1], hs2, dinvcol,
                  b2[None, :], g2[None, :], be2[None, :])

    starts = jnp.searchsorted(ibatch, jnp.arange(G + 1, dtype=_i32)).astype(_i32)
    starts = jnp.concatenate([starts, jnp.full((7,), N, _i32)])
    ib_pad = jnp.pad(ibatch, (0, NP - N), constant_values=G - 1)

    out = _segmax_kernel_fn()(r2, ib_pad, starts, st2.reshape(2 * D))
    return out


# speedup vs baseline: 26.3016x; 1.0740x over previous
import functools

import jax
import jax.numpy as jnp
from jax import lax
from jax.experimental import pallas as pl
from jax.experimental.pallas import tpu as pltpu
from jax.experimental.pallas import tpu_sc as plsc

N = 10000
NR = 10080
E = 320000
D = 128
G = 256
EPS = 1e-5

NC, NS = 2, 16

BLK = 400
NBLK = N // BLK

_f32 = jnp.float32
_i32 = jnp.int32

@functools.cache
def _mesh():
    return plsc.VectorSubcoreMesh(core_axis_name="c", subcore_axis_name="s",
                                  num_cores=NC, num_subcores=NS)


ER = E // 128
_MK = 128


def _worker_rows(w):
    off = jnp.where(w < 24, 80 * w, 1920 + 72 * (w - 24))
    n = jnp.where(w < 24, 80, jnp.where(w == 31, 76, 72))
    return off, n


def _load_idx_rows(src2d_hbm, buf, w, off):
    pltpu.sync_copy(src2d_hbm.at[pl.ds(off, 72)], buf.at[pl.ds(0, 72)])

    @pl.when(w < 24)
    def _():
        pltpu.sync_copy(src2d_hbm.at[pl.ds(off + 72, 8)], buf.at[pl.ds(72, 8)])

    @pl.when(w == 31)
    def _():
        pltpu.sync_copy(src2d_hbm.at[pl.ds(off + 72, 4)], buf.at[pl.ds(72, 4)])


_RT0 = 624
_RT15 = N - 15 * _RT0


@functools.cache
def _deg_kernel_fn():
    return pl.kernel(
        _deg_body,
        out_type=jax.ShapeDtypeStruct((NC * N,), _f32),
        mesh=_mesh(),
        scratch_types=[
            pltpu.VMEM_SHARED((N,), _f32),
            pltpu.VMEM((80, _MK), _i32),
            pltpu.VMEM((_MK,), _f32),
            pltpu.VMEM((_RT15,), _f32),
            pltpu.SemaphoreType.DMA,
        ],
    )


def _deg_body(dst2d_hbm, out_hbm, acc, idxv, onesv, zv, sem):
    c = lax.axis_index("c")
    s = lax.axis_index("s")
    wid = c * NS + s
    off, n = _worker_rows(wid)

    def _zs(i, _):
        zv[pl.ds(i * 16, 16)] = jnp.zeros((16,), _f32)
        return 0
    lax.fori_loop(0, _RT15 // 16, _zs, 0)

    @pl.when(s < 15)
    def _():
        pltpu.sync_copy(zv.at[pl.ds(0, _RT0)], acc.at[pl.ds(s * _RT0, _RT0)])

    @pl.when(s == 15)
    def _():
        pltpu.sync_copy(zv, acc.at[pl.ds(15 * _RT0, _RT15)])

    def _os(i, _):
        onesv[pl.ds(i * 16, 16)] = jnp.ones((16,), _f32)
        return 0
    lax.fori_loop(0, _MK // 16, _os, 0)
    _load_idx_rows(dst2d_hbm, idxv, wid, off)
    plsc.subcore_barrier()

    def _fire(j, _):
        pltpu.async_copy(onesv, acc.at[idxv.at[j]], sem, add=True)
        return 0
    lax.fori_loop(0, n, _fire, 0)

    def _drain(j, _):
        pltpu.make_async_copy(dst2d_hbm.at[pl.ds(0, 1)],
                              idxv.at[pl.ds(0, 1)], sem).wait()
        return 0
    lax.fori_loop(0, n, _drain, 0)
    plsc.subcore_barrier()

    @pl.when(s < 15)
    def _():
        pltpu.sync_copy(acc.at[pl.ds(s * _RT0, _RT0)], zv.at[pl.ds(0, _RT0)])
        pltpu.sync_copy(zv.at[pl.ds(0, _RT0)],
                        out_hbm.at[pl.ds(c * N + s * _RT0, _RT0)])

    @pl.when(s == 15)
    def _():
        pltpu.sync_copy(acc.at[pl.ds(15 * _RT0, _RT15)], zv)
        pltpu.sync_copy(zv, out_hbm.at[pl.ds(c * N + 15 * _RT0, _RT15)])


_NBUF = 2


@functools.cache
def _msg_kernel_fn():
    return pl.kernel(
        _msg_body,
        out_type=jax.ShapeDtypeStruct((NC, N, D), _f32),
        mesh=_mesh(),
        scratch_types=[
            pltpu.VMEM_SHARED((N, D), _f32),
            [pltpu.VMEM((_MK, D), _f32)] * _NBUF,
            [pltpu.VMEM((_MK,), _i32)] * _NBUF,
            pltpu.VMEM((80, _MK), _i32),
            pltpu.VMEM((8, D), _f32),
            [pltpu.SemaphoreType.DMA] * _NBUF,
            [pltpu.SemaphoreType.DMA] * _NBUF,
            [pltpu.SemaphoreType.DMA] * _NBUF,
        ],
    )


def _msg_body(hs_hbm, src_hbm, dst2d_hbm, out_hbm, acc, rows, srcb, dstv,
              zbuf, sem_g, sem_s, sem_i):
    c = lax.axis_index("c")
    s = lax.axis_index("s")
    wid = c * NS + s
    off, n = _worker_rows(wid)

    def _zrow(i, _):
        for cc in range(8):
            zbuf[i, pl.ds(cc * 16, 16)] = jnp.zeros((16,), _f32)
        return 0
    lax.fori_loop(0, 8, _zrow, 0)
    rbase = jnp.where(s < 15, s * _RT0, 15 * _RT0)
    ncop = jnp.where(s < 15, _RT0 // 8, _RT15 // 8)

    def _zc(b, _):
        pltpu.sync_copy(zbuf, acc.at[pl.ds(rbase + b * 8, 8)])
        return 0
    lax.fori_loop(0, ncop, _zc, 0)
    _load_idx_rows(dst2d_hbm, dstv, wid, off)
    plsc.subcore_barrier()

    ebase = off * _MK

    for i in range(_NBUF):
        pltpu.sync_copy(src_hbm.at[pl.ds(ebase + i * _MK, _MK)], srcb[i])
        pltpu.async_copy(hs_hbm.at[srcb[i]], rows[i], sem_g[i])

    def _group(cg, _):
        for i in range(_NBUF):
            k = cg * _NBUF + i

            @pl.when(k < n)
            def _():
                pltpu.make_async_copy(hs_hbm.at[pl.ds(0, _MK)], rows[i],
                                      sem_g[i]).wait()

                @pl.when(k + _NBUF < n)
                def _():
                    pltpu.async_copy(
                        src_hbm.at[pl.ds(ebase + (k + _NBUF) * _MK, _MK)],
                        srcb[i], sem_i[i])
                desc = pltpu.async_copy(rows[i], acc.at[dstv.at[k]],
                                        sem_s[i], add=True)

                @pl.when(k + _NBUF < n)
                def _():
                    desc.wait()
                    pltpu.make_async_copy(
                        src_hbm.at[pl.ds(0, _MK)], srcb[i], sem_i[i]).wait()
                    pltpu.async_copy(hs_hbm.at[srcb[i]], rows[i], sem_g[i])
        return 0
    lax.fori_loop(0, 40, _group, 0)

    for i in range(_NBUF):
        pltpu.make_async_copy(hs_hbm.at[pl.ds(0, _MK)], rows[i],
                              sem_s[i]).wait()
    plsc.subcore_barrier()

    @pl.when(s < 15)
    def _():
        pltpu.sync_copy(acc.at[pl.ds(s * _RT0, _RT0)],
                        out_hbm.at[c, pl.ds(s * _RT0, _RT0)])

    @pl.when(s == 15)
    def _():
        pltpu.sync_copy(acc.at[pl.ds(15 * _RT0, _RT15)],
                        out_hbm.at[c, pl.ds(15 * _RT0, _RT15)])


_SC_C = 64


@functools.cache
def _segmax_kernel_fn():
    return pl.kernel(
        _segmax_body,
        out_type=jax.ShapeDtypeStruct((G, D), _f32),
        mesh=_mesh(),
        scratch_types=[
            pltpu.VMEM((8 * D,), _f32),
            pltpu.VMEM((_SC_C + 8, D), _f32),
            pltpu.VMEM((96,), _i32),
            pltpu.VMEM((256,), _f32),
            pltpu.VMEM((16,), _i32),
            pltpu.VMEM((8, D), _f32),
        ],
    )


def _segmax_body(r_hbm, ib_hbm, starts_hbm, st_hbm, out_hbm,
                 accv, rowbuf, ibv, stv, sv, obuf):
    c = lax.axis_index("c")
    s = lax.axis_index("s")
    wid = c * NS + s
    g0 = wid * 8

    pltpu.sync_copy(starts_hbm.at[pl.ds(g0, 16)], sv)
    pltpu.sync_copy(st_hbm, stv)
    svv = sv[pl.ds(0, 16)]
    start = svv[0]
    end = svv[8]

    def _ini(i, _):
        accv[pl.ds(i * 16, 16)] = jnp.full((16,), -jnp.inf, _f32)
        return 0
    lax.fori_loop(0, 8 * D // 16, _ini, 0)

    nch = lax.div(end - start + (_SC_C - 1), _SC_C)

    def _chunk(t, _):
        base = start + t * _SC_C
        nrows = jnp.minimum(_SC_C, end - base)
        ab = lax.div(base, 8) * 8
        sh = base - ab
        pltpu.sync_copy(r_hbm.at[pl.ds(ab, _SC_C + 8)], rowbuf)
        pltpu.sync_copy(ib_hbm.at[pl.ds(ab, 80)], ibv.at[pl.ds(0, 80)])

        def _row(i, _):
            ib = ibv[pl.ds(sh + i, 16)][0]
            off = (ib - g0) * D
            for cc in range(8):
                a = accv[pl.ds(off + cc * 16, 16)]
                v = rowbuf[sh + i, pl.ds(cc * 16, 16)]
                accv[pl.ds(off + cc * 16, 16)] = jnp.maximum(a, v)
            return 0
        lax.fori_loop(0, nrows, _row, 0)
        return 0
    lax.fori_loop(0, nch, _chunk, 0)

    for k in range(8):
        for cc in range(8):
            s2 = stv[pl.ds(cc * 16, 16)]
            t2 = stv[pl.ds(D + cc * 16, 16)]
            obuf[k, pl.ds(cc * 16, 16)] = accv[pl.ds(k * D + cc * 16, 16)] * s2 + t2
    pltpu.sync_copy(obuf, out_hbm.at[pl.ds(g0, 8)])


def _k1_body(x_ref, deg_ref, w_ref, hs_ref, dinv_ref):
    dinv = lax.rsqrt(deg_ref[...])
    h = jnp.dot(x_ref[...], w_ref[...], preferred_element_type=_f32)
    hs_ref[...] = h * dinv
    dinv_ref[...] = dinv


def _k1(x, degcol, W):
    return pl.pallas_call(
        _k1_body,
        grid=(NBLK,),
        in_specs=[
            pl.BlockSpec((BLK, D), lambda i: (i, 0)),
            pl.BlockSpec((BLK, 1), lambda i: (i, 0)),
            pl.BlockSpec((D, D), lambda i: (0, 0)),
        ],
        out_specs=[
            pl.BlockSpec((BLK, D), lambda i: (i, 0)),
            pl.BlockSpec((BLK, 1), lambda i: (i, 0)),
        ],
        out_shape=[
            jax.ShapeDtypeStruct((N, D), _f32),
            jax.ShapeDtypeStruct((N, 1), _f32),
        ],
    )(x, degcol, W)


def _k23_body(p0_ref, p1_ref, hs_ref, dinv_ref, b_ref, g_ref, be_ref, w_ref,
              hs2_ref, rbuf, ssum, ssq, stscr):
    t = pl.program_id(0)
    i = pl.program_id(1)

    @pl.when(t == 0)
    def _():
        v = (dinv_ref[...] * (p0_ref[...] + p1_ref[...] + hs_ref[...])
             + b_ref[...])
        r = jnp.maximum(v, 0.0)
        rbuf[pl.ds(i * BLK, BLK), :] = r
        cs = jnp.sum(r, axis=0, keepdims=True)
        cq = jnp.sum(r * r, axis=0, keepdims=True)

        @pl.when(i == 0)
        def _():
            ssum[...] = cs
            ssq[...] = cq

        @pl.when(i > 0)
        def _():
            ssum[...] += cs
            ssq[...] += cq

        @pl.when(i == NBLK - 1)
        def _():
            mean = ssum[...] / N
            var = ssq[...] / N - mean * mean
            sc = g_ref[...] * lax.rsqrt(var + EPS)
            stscr[...] = jnp.concatenate(
                [sc, be_ref[...] - mean * sc], axis=0)

    @pl.when(t == 1)
    def _():
        sc = stscr[0:1, :]
        sh = stscr[1:2, :]
        x2 = rbuf[pl.ds(i * BLK, BLK), :] * sc + sh
        h2 = jnp.dot(x2, w_ref[...], preferred_element_type=_f32)
        hs2_ref[...] = h2 * dinv_ref[...]


def _k23(p0, p1, hs, dinvcol, b, g, be, W2):
    blk_p0 = pl.BlockSpec((BLK, D), lambda t, i: (i * (1 - t), 0))
    return pl.pallas_call(
        _k23_body,
        grid=(2, NBLK),
        in_specs=[
            blk_p0,
            blk_p0,
            blk_p0,
            pl.BlockSpec((BLK, 1), lambda t, i: (i, 0)),
            pl.BlockSpec((1, D), lambda t, i: (0, 0)),
            pl.BlockSpec((1, D), lambda t, i: (0, 0)),
            pl.BlockSpec((1, D), lambda t, i: (0, 0)),
            pl.BlockSpec((D, D), lambda t, i: (0, 0)),
        ],
        out_specs=pl.BlockSpec((BLK, D), lambda t, i: (i * t, 0)),
        out_shape=jax.ShapeDtypeStruct((N, D), _f32),
        scratch_shapes=[
            pltpu.VMEM((N, D), _f32),
            pltpu.VMEM((1, D), _f32),
            pltpu.VMEM((1, D), _f32),
            pltpu.VMEM((2, D), _f32),
        ],
    )(p0, p1, hs, dinvcol, b, g, be, W2)


def _k2_body(p0_ref, p1_ref, hs_ref, dinv_ref, b_ref, g_ref, be_ref,
             r_ref, st_ref, ssum, ssq):
    i = pl.program_id(0)
    v = dinv_ref[...] * (p0_ref[...] + p1_ref[...] + hs_ref[...]) + b_ref[...]
    r = jnp.maximum(v, 0.0)
    r_ref[...] = r
    cs = jnp.sum(r, axis=0, keepdims=True)
    cq = jnp.sum(r * r, axis=0, keepdims=True)

    @pl.when(i == 0)
    def _():
        ssum[...] = cs
        ssq[...] = cq

    @pl.when(i > 0)
    def _():
        ssum[...] += cs
        ssq[...] += cq

    @pl.when(i == NBLK - 1)
    def _():
        mean = ssum[...] / N
        var = ssq[...] / N - mean * mean
        sc = g_ref[...] * lax.rsqrt(var + EPS)
        st_ref[...] = jnp.concatenate([sc, be_ref[...] - mean * sc], axis=0)


def _k2(p0, p1, hs, dinvcol, b, g, be):
    return pl.pallas_call(
        _k2_body,
        grid=(NBLK,),
        in_specs=[
            pl.BlockSpec((BLK, D), lambda i: (i, 0)),
            pl.BlockSpec((BLK, D), lambda i: (i, 0)),
            pl.BlockSpec((BLK, D), lambda i: (i, 0)),
            pl.BlockSpec((BLK, 1), lambda i: (i, 0)),
            pl.BlockSpec((1, D), lambda i: (0, 0)),
            pl.BlockSpec((1, D), lambda i: (0, 0)),
            pl.BlockSpec((1, D), lambda i: (0, 0)),
        ],
        out_specs=[
            pl.BlockSpec((BLK, D), lambda i: (i, 0)),
            pl.BlockSpec((2, D), lambda i: (0, 0)),
        ],
        out_shape=[
            jax.ShapeDtypeStruct((NR, D), _f32),
            jax.ShapeDtypeStruct((2, D), _f32),
        ],
        scratch_shapes=[
            pltpu.VMEM((1, D), _f32),
            pltpu.VMEM((1, D), _f32),
        ],
    )(p0, p1, hs, dinvcol, b, g, be)


def kernel(input_feature, input_adj, ibatch, W1, b1, g1, be1, W2, b2, g2, be2):
    src1d = input_adj[0]
    dst2d = input_adj[1].reshape(ER, 128)

    degp = _deg_kernel_fn()(dst2d).reshape(NC, N)
    degcol = (degp[0] + degp[1] + 1.0)[:, None]

    hs1, dinvcol = _k1(input_feature, degcol, W1)

    parts1 = _msg_kernel_fn()(hs1, src1d, dst2d)
    hs2 = _k23(parts1[0], parts1[1], hs1, dinvcol,
               b1[None, :], g1[None, :], be1[None, :], W2)

    parts2 = _msg_kernel_fn()(hs2, src1d, dst2d)
    r2, st2 = _k2(parts2[0], parts2[1], hs2, dinvcol,
                  b2[None, :], g2[None, :], be2[None, :])

    starts = jnp.searchsorted(ibatch, jnp.arange(G + 1, dtype=_i32)).astype(_i32)
    starts = jnp.concatenate([starts, jnp.full((7,), N, _i32)])
    ib_pad = jnp.pad(ibatch, (0, NR - N), constant_values=G - 1)

    out = _segmax_kernel_fn()(r2, ib_pad, starts, st2.reshape(2 * D))
    return out


# speedup vs baseline: 27.2802x; 1.0372x over previous
import functools

import jax
import jax.numpy as jnp
from jax import lax
from jax.experimental import pallas as pl
from jax.experimental.pallas import tpu as pltpu
from jax.experimental.pallas import tpu_sc as plsc

N = 10000
NR = 10080
E = 320000
D = 128
G = 256
EPS = 1e-5

NC, NS = 2, 16

BLK = 400
NBLK = N // BLK

_f32 = jnp.float32
_i32 = jnp.int32

@functools.cache
def _mesh():
    return plsc.VectorSubcoreMesh(core_axis_name="c", subcore_axis_name="s",
                                  num_cores=NC, num_subcores=NS)


ER = E // 128
_MK = 128


def _worker_rows(w):
    off = jnp.where(w < 24, 80 * w, 1920 + 72 * (w - 24))
    n = jnp.where(w < 24, 80, jnp.where(w == 31, 76, 72))
    return off, n


def _load_idx_rows(src2d_hbm, buf, w, off):
    pltpu.sync_copy(src2d_hbm.at[pl.ds(off, 72)], buf.at[pl.ds(0, 72)])

    @pl.when(w < 24)
    def _():
        pltpu.sync_copy(src2d_hbm.at[pl.ds(off + 72, 8)], buf.at[pl.ds(72, 8)])

    @pl.when(w == 31)
    def _():
        pltpu.sync_copy(src2d_hbm.at[pl.ds(off + 72, 4)], buf.at[pl.ds(72, 4)])


_RT0 = 624
_RT15 = N - 15 * _RT0


def _deg_rows(t):
    off = jnp.where(t < 8, 160 * t, 1280 + 152 * (t - 8))
    n = jnp.where(t < 8, 160, jnp.where(t == 15, 156, 152))
    return off, n


@functools.cache
def _deg_kernel_fn():
    return pl.kernel(
        _deg_body,
        out_type=(jax.ShapeDtypeStruct((N,), _f32),
                  jax.ShapeDtypeStruct((272,), _i32)),
        mesh=_mesh(),
        scratch_types=[
            pltpu.VMEM_SHARED((N,), _f32),
            pltpu.VMEM_SHARED((256,), _i32),
            pltpu.VMEM((160, _MK), _i32),
            pltpu.VMEM((_MK,), _f32),
            pltpu.VMEM((_MK,), _i32),
            pltpu.VMEM((112,), _i32),
            pltpu.VMEM((_RT15,), _f32),
            pltpu.VMEM((384,), _i32),
            pltpu.VMEM((384,), _i32),
            pltpu.SemaphoreType.DMA,
        ],
    )


def _deg_body(dst2d_hbm, ib_hbm, deg_hbm, starts_hbm, acc, hist, idxv,
              onesf, onesi, tailv, zv, stva, stvb, sem):
    c = lax.axis_index("c")
    s = lax.axis_index("s")

    def _zs(i, _):
        zv[pl.ds(i * 16, 16)] = jnp.zeros((16,), _f32)
        return 0
    lax.fori_loop(0, _RT15 // 16, _zs, 0)

    def _os(i, _):
        onesf[pl.ds(i * 16, 16)] = jnp.ones((16,), _f32)
        onesi[pl.ds(i * 16, 16)] = jnp.ones((16,), _i32)
        return 0
    lax.fori_loop(0, _MK // 16, _os, 0)

    @pl.when(c == 1)
    def _():
        @pl.when(s < 15)
        def _():
            pltpu.sync_copy(zv.at[pl.ds(0, _RT0)],
                            acc.at[pl.ds(s * _RT0, _RT0)])

        @pl.when(s == 15)
        def _():
            pltpu.sync_copy(zv, acc.at[pl.ds(15 * _RT0, _RT15)])

        off, _n = _deg_rows(s)
        pltpu.sync_copy(dst2d_hbm.at[pl.ds(off, 152)],
                        idxv.at[pl.ds(0, 152)])

        @pl.when(s < 8)
        def _():
            pltpu.sync_copy(dst2d_hbm.at[pl.ds(off + 152, 8)],
                            idxv.at[pl.ds(152, 8)])

        @pl.when(s == 15)
        def _():
            pltpu.sync_copy(dst2d_hbm.at[pl.ds(off + 152, 4)],
                            idxv.at[pl.ds(152, 4)])

    @pl.when(c == 0)
    def _():
        @pl.when(s == 0)
        def _():
            def _zh(i, _):
                stva[pl.ds(i * 16, 16)] = jnp.zeros((16,), _i32)
                return 0
            lax.fori_loop(0, 16, _zh, 0)
            pltpu.sync_copy(stva.at[pl.ds(0, 256)], hist)

        ibase = jnp.where(s < 15, s * _RT0, 15 * _RT0)
        nfull = jnp.where(s < 15, 4, 5)

        def _ldc(j, _):
            pltpu.sync_copy(ib_hbm.at[pl.ds(ibase + j * _MK, _MK)],
                            idxv.at[j])
            return 0
        lax.fori_loop(0, nfull, _ldc, 0)

        @pl.when(s < 15)
        def _():
            pltpu.sync_copy(ib_hbm.at[pl.ds(ibase + 4 * _MK, 112)], tailv)

    plsc.subcore_barrier()

    @pl.when(c == 1)
    def _():
        _off, n = _deg_rows(s)

        def _fire(j, _):
            pltpu.async_copy(onesf, acc.at[idxv.at[j]], sem, add=True)
            return 0
        lax.fori_loop(0, n, _fire, 0)

        def _drain(j, _):
            pltpu.make_async_copy(dst2d_hbm.at[pl.ds(0, 1)],
                                  idxv.at[pl.ds(0, 1)], sem).wait()
            return 0
        lax.fori_loop(0, n, _drain, 0)

    @pl.when(c == 0)
    def _():
        nfull = jnp.where(s < 15, 4, 5)

        def _fire(j, _):
            pltpu.async_copy(onesi, hist.at[idxv.at[j]], sem, add=True)
            return 0
        lax.fori_loop(0, nfull, _fire, 0)

        @pl.when(s < 15)
        def _():
            pltpu.async_copy(onesi.at[pl.ds(0, 112)], hist.at[tailv], sem,
                             add=True)

        def _drain(j, _):
            pltpu.make_async_copy(dst2d_hbm.at[pl.ds(0, 1)],
                                  idxv.at[pl.ds(0, 1)], sem).wait()
            return 0
        lax.fori_loop(0, nfull, _drain, 0)

        @pl.when(s < 15)
        def _():
            pltpu.make_async_copy(ib_hbm.at[pl.ds(0, 112)], tailv, sem).wait()

    plsc.subcore_barrier()

    @pl.when(c == 1)
    def _():
        @pl.when(s < 15)
        def _():
            pltpu.sync_copy(acc.at[pl.ds(s * _RT0, _RT0)],
                            zv.at[pl.ds(0, _RT0)])
            pltpu.sync_copy(zv.at[pl.ds(0, _RT0)],
                            deg_hbm.at[pl.ds(s * _RT0, _RT0)])

        @pl.when(s == 15)
        def _():
            pltpu.sync_copy(acc.at[pl.ds(15 * _RT0, _RT15)], zv)
            pltpu.sync_copy(zv, deg_hbm.at[pl.ds(15 * _RT0, _RT15)])

    @pl.when(jnp.logical_and(c == 0, s == 0))
    def _():
        def _zp(i, _):
            stva[pl.ds(i * 16, 16)] = jnp.zeros((16,), _i32)
            stvb[pl.ds(i * 16, 16)] = jnp.zeros((16,), _i32)
            return 0
        lax.fori_loop(0, 8, _zp, 0)
        pltpu.sync_copy(hist, stva.at[pl.ds(128, 256)])

        bufs = (stva, stvb)
        for r, k in enumerate((1, 2, 4, 8, 16, 32, 64, 128)):
            srcb = bufs[r % 2]
            dstb = bufs[1 - r % 2]
            for i in range(16):
                dstb[pl.ds(128 + 16 * i, 16)] = (
                    srcb[pl.ds(128 + 16 * i, 16)]
                    + srcb[pl.ds(128 + 16 * i - k, 16)])
        for i in range(16):
            stvb[pl.ds(16 * i, 16)] = stva[pl.ds(127 + 16 * i, 16)]
        stvb[pl.ds(256, 16)] = jnp.full((16,), N, _i32)
        pltpu.sync_copy(stvb.at[pl.ds(0, 272)], starts_hbm)


_NBUF = 2


@functools.cache
def _msg_kernel_fn():
    return pl.kernel(
        _msg_body,
        out_type=jax.ShapeDtypeStruct((NC, N, D), _f32),
        mesh=_mesh(),
        scratch_types=[
            pltpu.VMEM_SHARED((N, D), _f32),
            [pltpu.VMEM((_MK, D), _f32)] * _NBUF,
            [pltpu.VMEM((_MK,), _i32)] * _NBUF,
            pltpu.VMEM((80, _MK), _i32),
            pltpu.VMEM((8, D), _f32),
            [pltpu.SemaphoreType.DMA] * _NBUF,
            [pltpu.SemaphoreType.DMA] * _NBUF,
            [pltpu.SemaphoreType.DMA] * _NBUF,
        ],
    )


def _msg_body(hs_hbm, src_hbm, dst2d_hbm, out_hbm, acc, rows, srcb, dstv,
              zbuf, sem_g, sem_s, sem_i):
    c = lax.axis_index("c")
    s = lax.axis_index("s")
    wid = c * NS + s
    off, n = _worker_rows(wid)

    def _zrow(i, _):
        for cc in range(8):
            zbuf[i, pl.ds(cc * 16, 16)] = jnp.zeros((16,), _f32)
        return 0
    lax.fori_loop(0, 8, _zrow, 0)
    rbase = jnp.where(s < 15, s * _RT0, 15 * _RT0)
    ncop = jnp.where(s < 15, _RT0 // 8, _RT15 // 8)

    def _zc(b, _):
        pltpu.sync_copy(zbuf, acc.at[pl.ds(rbase + b * 8, 8)])
        return 0
    lax.fori_loop(0, ncop, _zc, 0)
    _load_idx_rows(dst2d_hbm, dstv, wid, off)
    plsc.subcore_barrier()

    ebase = off * _MK

    for i in range(_NBUF):
        pltpu.sync_copy(src_hbm.at[pl.ds(ebase + i * _MK, _MK)], srcb[i])
        pltpu.async_copy(hs_hbm.at[srcb[i]], rows[i], sem_g[i])

    def _group(cg, _):
        for i in range(_NBUF):
            k = cg * _NBUF + i

            @pl.when(k < n)
            def _():
                pltpu.make_async_copy(hs_hbm.at[pl.ds(0, _MK)], rows[i],
                                      sem_g[i]).wait()

                @pl.when(k + _NBUF < n)
                def _():
                    pltpu.async_copy(
                        src_hbm.at[pl.ds(ebase + (k + _NBUF) * _MK, _MK)],
                        srcb[i], sem_i[i])
                desc = pltpu.async_copy(rows[i], acc.at[dstv.at[k]],
                                        sem_s[i], add=True)

                @pl.when(k + _NBUF < n)
                def _():
                    desc.wait()
                    pltpu.make_async_copy(
                        src_hbm.at[pl.ds(0, _MK)], srcb[i], sem_i[i]).wait()
                    pltpu.async_copy(hs_hbm.at[srcb[i]], rows[i], sem_g[i])
        return 0
    lax.fori_loop(0, 40, _group, 0)

    for i in range(_NBUF):
        pltpu.make_async_copy(hs_hbm.at[pl.ds(0, _MK)], rows[i],
                              sem_s[i]).wait()
    plsc.subcore_barrier()

    @pl.when(s < 15)
    def _():
        pltpu.sync_copy(acc.at[pl.ds(s * _RT0, _RT0)],
                        out_hbm.at[c, pl.ds(s * _RT0, _RT0)])

    @pl.when(s == 15)
    def _():
        pltpu.sync_copy(acc.at[pl.ds(15 * _RT0, _RT15)],
                        out_hbm.at[c, pl.ds(15 * _RT0, _RT15)])


_SC_C = 64


@functools.cache
def _segmax_kernel_fn():
    return pl.kernel(
        _segmax_body,
        out_type=jax.ShapeDtypeStruct((G, D), _f32),
        mesh=_mesh(),
        scratch_types=[
            pltpu.VMEM((_SC_C + 8, D), _f32),
            pltpu.VMEM((256,), _f32),
            pltpu.VMEM((16,), _i32),
            pltpu.VMEM((8, D), _f32),
        ],
    )


def _segmax_body(r_hbm, starts_hbm, st_hbm, out_hbm, rowbuf, stv, sv, obuf):
    c = lax.axis_index("c")
    s = lax.axis_index("s")
    wid = c * NS + s
    g0 = wid * 8

    pltpu.sync_copy(starts_hbm.at[pl.ds(g0, 16)], sv)
    pltpu.sync_copy(st_hbm, stv)
    svv = sv[pl.ds(0, 16)]

    ninf = jnp.full((16,), -jnp.inf, _f32)
    for k in range(8):
        gs = svv[k]
        ge = svv[k + 1]
        nch = lax.div(ge - gs + (_SC_C - 1), _SC_C)

        def _chunk(t, carry):
            base = gs + t * _SC_C
            nrows = jnp.minimum(_SC_C, ge - base)
            ab = jnp.minimum(lax.div(base, 8) * 8, N - (_SC_C + 8))
            sh = base - ab
            pltpu.sync_copy(r_hbm.at[pl.ds(ab, _SC_C + 8)], rowbuf)

            def _row(i, acc8):
                return tuple(
                    jnp.maximum(acc8[cc], rowbuf[sh + i, pl.ds(cc * 16, 16)])
                    for cc in range(8))
            return lax.fori_loop(0, nrows, _row, carry)
        acc8 = lax.fori_loop(0, nch, _chunk, (ninf,) * 8)

        for cc in range(8):
            s2 = stv[pl.ds(cc * 16, 16)]
            t2 = stv[pl.ds(D + cc * 16, 16)]
            obuf[k, pl.ds(cc * 16, 16)] = acc8[cc] * s2 + t2
    pltpu.sync_copy(obuf, out_hbm.at[pl.ds(g0, 8)])


def _k1_body(x_ref, deg_ref, w_ref, hs_ref, dinv_ref):
    dinv = lax.rsqrt(deg_ref[...] + 1.0)
    h = jnp.dot(x_ref[...], w_ref[...], preferred_element_type=_f32)
    hs_ref[...] = h * dinv
    dinv_ref[...] = dinv


def _k1(x, degcol, W):
    return pl.pallas_call(
        _k1_body,
        grid=(NBLK,),
        in_specs=[
            pl.BlockSpec((BLK, D), lambda i: (i, 0)),
            pl.BlockSpec((BLK, 1), lambda i: (i, 0)),
            pl.BlockSpec((D, D), lambda i: (0, 0)),
        ],
        out_specs=[
            pl.BlockSpec((BLK, D), lambda i: (i, 0)),
            pl.BlockSpec((BLK, 1), lambda i: (i, 0)),
        ],
        out_shape=[
            jax.ShapeDtypeStruct((N, D), _f32),
            jax.ShapeDtypeStruct((N, 1), _f32),
        ],
    )(x, degcol, W)


def _k23_body(p0_ref, p1_ref, hs_ref, dinv_ref, b_ref, g_ref, be_ref, w_ref,
              hs2_ref, rbuf, ssum, ssq, stscr):
    t = pl.program_id(0)
    i = pl.program_id(1)

    @pl.when(t == 0)
    def _():
        v = (dinv_ref[...] * (p0_ref[...] + p1_ref[...] + hs_ref[...])
             + b_ref[...])
        r = jnp.maximum(v, 0.0)
        rbuf[pl.ds(i * BLK, BLK), :] = r
        cs = jnp.sum(r, axis=0, keepdims=True)
        cq = jnp.sum(r * r, axis=0, keepdims=True)

        @pl.when(i == 0)
        def _():
            ssum[...] = cs
            ssq[...] = cq

        @pl.when(i > 0)
        def _():
            ssum[...] += cs
            ssq[...] += cq

        @pl.when(i == NBLK - 1)
        def _():
            mean = ssum[...] / N
            var = ssq[...] / N - mean * mean
            sc = g_ref[...] * lax.rsqrt(var + EPS)
            stscr[...] = jnp.concatenate(
                [sc, be_ref[...] - mean * sc], axis=0)

    @pl.when(t == 1)
    def _():
        sc = stscr[0:1, :]
        sh = stscr[1:2, :]
        x2 = rbuf[pl.ds(i * BLK, BLK), :] * sc + sh
        h2 = jnp.dot(x2, w_ref[...], preferred_element_type=_f32)
        hs2_ref[...] = h2 * dinv_ref[...]


def _k23(p0, p1, hs, dinvcol, b, g, be, W2):
    blk_p0 = pl.BlockSpec((BLK, D), lambda t, i: (i * (1 - t), 0))
    return pl.pallas_call(
        _k23_body,
        grid=(2, NBLK),
        in_specs=[
            blk_p0,
            blk_p0,
            blk_p0,
            pl.BlockSpec((BLK, 1), lambda t, i: (i, 0)),
            pl.BlockSpec((1, D), lambda t, i: (0, 0)),
            pl.BlockSpec((1, D), lambda t, i: (0, 0)),
            pl.BlockSpec((1, D), lambda t, i: (0, 0)),
            pl.BlockSpec((D, D), lambda t, i: (0, 0)),
        ],
        out_specs=pl.BlockSpec((BLK, D), lambda t, i: (i * t, 0)),
        out_shape=jax.ShapeDtypeStruct((N, D), _f32),
        scratch_shapes=[
            pltpu.VMEM((N, D), _f32),
            pltpu.VMEM((1, D), _f32),
            pltpu.VMEM((1, D), _f32),
            pltpu.VMEM((2, D), _f32),
        ],
    )(p0, p1, hs, dinvcol, b, g, be, W2)


def _k2_body(p0_ref, p1_ref, hs_ref, dinv_ref, b_ref, g_ref, be_ref,
             r_ref, st_ref, ssum, ssq):
    i = pl.program_id(0)
    v = dinv_ref[...] * (p0_ref[...] + p1_ref[...] + hs_ref[...]) + b_ref[...]
    r = jnp.maximum(v, 0.0)
    r_ref[...] = r
    cs = jnp.sum(r, axis=0, keepdims=True)
    cq = jnp.sum(r * r, axis=0, keepdims=True)

    @pl.when(i == 0)
    def _():
        ssum[...] = cs
        ssq[...] = cq

    @pl.when(i > 0)
    def _():
        ssum[...] += cs
        ssq[...] += cq

    @pl.when(i == NBLK - 1)
    def _():
        mean = ssum[...] / N
        var = ssq[...] / N - mean * mean
        sc = g_ref[...] * lax.rsqrt(var + EPS)
        st_ref[...] = jnp.concatenate([sc, be_ref[...] - mean * sc], axis=0)


def _k2(p0, p1, hs, dinvcol, b, g, be):
    return pl.pallas_call(
        _k2_body,
        grid=(NBLK,),
        in_specs=[
            pl.BlockSpec((BLK, D), lambda i: (i, 0)),
            pl.BlockSpec((BLK, D), lambda i: (i, 0)),
            pl.BlockSpec((BLK, D), lambda i: (i, 0)),
            pl.BlockSpec((BLK, 1), lambda i: (i, 0)),
            pl.BlockSpec((1, D), lambda i: (0, 0)),
            pl.BlockSpec((1, D), lambda i: (0, 0)),
            pl.BlockSpec((1, D), lambda i: (0, 0)),
        ],
        out_specs=[
            pl.BlockSpec((BLK, D), lambda i: (i, 0)),
            pl.BlockSpec((2, D), lambda i: (0, 0)),
        ],
        out_shape=[
            jax.ShapeDtypeStruct((N, D), _f32),
            jax.ShapeDtypeStruct((2, D), _f32),
        ],
        scratch_shapes=[
            pltpu.VMEM((1, D), _f32),
            pltpu.VMEM((1, D), _f32),
        ],
    )(p0, p1, hs, dinvcol, b, g, be)


def kernel(input_feature, input_adj, ibatch, W1, b1, g1, be1, W2, b2, g2, be2):
    src1d = input_adj[0]
    dst2d = input_adj[1].reshape(ER, 128)

    deg, starts = _deg_kernel_fn()(dst2d, ibatch)
    hs1, dinvcol = _k1(input_feature, deg[:, None], W1)

    parts1 = _msg_kernel_fn()(hs1, src1d, dst2d)
    hs2 = _k23(parts1[0], parts1[1], hs1, dinvcol,
               b1[None, :], g1[None, :], be1[None, :], W2)

    parts2 = _msg_kernel_fn()(hs2, src1d, dst2d)
    r2, st2 = _k2(parts2[0], parts2[1], hs2, dinvcol,
                  b2[None, :], g2[None, :], be2[None, :])

    out = _segmax_kernel_fn()(r2, starts, st2.reshape(2 * D))
    return out


# speedup vs baseline: 27.5612x; 1.0103x over previous
import functools

import jax
import jax.numpy as jnp
from jax import lax
from jax.experimental import pallas as pl
from jax.experimental.pallas import tpu as pltpu
from jax.experimental.pallas import tpu_sc as plsc

N = 10000
NR = 10080
E = 320000
D = 128
G = 256
EPS = 1e-5

NC, NS = 2, 16

BLK = 400
NBLK = N // BLK

_f32 = jnp.float32
_i32 = jnp.int32

@functools.cache
def _mesh():
    return plsc.VectorSubcoreMesh(core_axis_name="c", subcore_axis_name="s",
                                  num_cores=NC, num_subcores=NS)


ER = E // 128
_MK = 128


def _worker_rows(w):
    off = jnp.where(w < 24, 80 * w, 1920 + 72 * (w - 24))
    n = jnp.where(w < 24, 80, jnp.where(w == 31, 76, 72))
    return off, n


def _load_idx_rows(src2d_hbm, buf, w, off):
    pltpu.sync_copy(src2d_hbm.at[pl.ds(off, 72)], buf.at[pl.ds(0, 72)])

    @pl.when(w < 24)
    def _():
        pltpu.sync_copy(src2d_hbm.at[pl.ds(off + 72, 8)], buf.at[pl.ds(72, 8)])

    @pl.when(w == 31)
    def _():
        pltpu.sync_copy(src2d_hbm.at[pl.ds(off + 72, 4)], buf.at[pl.ds(72, 4)])


_RT0 = 624
_RT15 = N - 15 * _RT0


def _deg_rows(t):
    off = jnp.where(t < 8, 160 * t, 1280 + 152 * (t - 8))
    n = jnp.where(t < 8, 160, jnp.where(t == 15, 156, 152))
    return off, n


@functools.cache
def _deg_kernel_fn():
    return pl.kernel(
        _deg_body,
        out_type=(jax.ShapeDtypeStruct((N,), _f32),
                  jax.ShapeDtypeStruct((272,), _i32)),
        mesh=_mesh(),
        scratch_types=[
            pltpu.VMEM_SHARED((N,), _f32),
            pltpu.VMEM_SHARED((256,), _i32),
            pltpu.VMEM((160, _MK), _i32),
            pltpu.VMEM((_MK,), _f32),
            pltpu.VMEM((_MK,), _i32),
            pltpu.VMEM((112,), _i32),
            pltpu.VMEM((_RT15,), _f32),
            pltpu.VMEM((384,), _i32),
            pltpu.VMEM((384,), _i32),
            pltpu.SemaphoreType.DMA,
        ],
    )


def _deg_body(dst2d_hbm, ib_hbm, deg_hbm, starts_hbm, acc, hist, idxv,
              onesf, onesi, tailv, zv, stva, stvb, sem):
    c = lax.axis_index("c")
    s = lax.axis_index("s")

    def _zs(i, _):
        zv[pl.ds(i * 16, 16)] = jnp.zeros((16,), _f32)
        return 0
    lax.fori_loop(0, _RT15 // 16, _zs, 0)

    def _os(i, _):
        onesf[pl.ds(i * 16, 16)] = jnp.ones((16,), _f32)
        onesi[pl.ds(i * 16, 16)] = jnp.ones((16,), _i32)
        return 0
    lax.fori_loop(0, _MK // 16, _os, 0)

    @pl.when(c == 1)
    def _():
        @pl.when(s < 15)
        def _():
            pltpu.sync_copy(zv.at[pl.ds(0, _RT0)],
                            acc.at[pl.ds(s * _RT0, _RT0)])

        @pl.when(s == 15)
        def _():
            pltpu.sync_copy(zv, acc.at[pl.ds(15 * _RT0, _RT15)])

        off, _n = _deg_rows(s)
        pltpu.sync_copy(dst2d_hbm.at[pl.ds(off, 152)],
                        idxv.at[pl.ds(0, 152)])

        @pl.when(s < 8)
        def _():
            pltpu.sync_copy(dst2d_hbm.at[pl.ds(off + 152, 8)],
                            idxv.at[pl.ds(152, 8)])

        @pl.when(s == 15)
        def _():
            pltpu.sync_copy(dst2d_hbm.at[pl.ds(off + 152, 4)],
                            idxv.at[pl.ds(152, 4)])

    @pl.when(c == 0)
    def _():
        @pl.when(s == 0)
        def _():
            def _zh(i, _):
                stva[pl.ds(i * 16, 16)] = jnp.zeros((16,), _i32)
                return 0
            lax.fori_loop(0, 16, _zh, 0)
            pltpu.sync_copy(stva.at[pl.ds(0, 256)], hist)

        ibase = jnp.where(s < 15, s * _RT0, 15 * _RT0)
        nfull = jnp.where(s < 15, 4, 5)

        def _ldc(j, _):
            pltpu.sync_copy(ib_hbm.at[pl.ds(ibase + j * _MK, _MK)],
                            idxv.at[j])
            return 0
        lax.fori_loop(0, nfull, _ldc, 0)

        @pl.when(s < 15)
        def _():
            pltpu.sync_copy(ib_hbm.at[pl.ds(ibase + 4 * _MK, 112)], tailv)

    plsc.subcore_barrier()

    @pl.when(c == 1)
    def _():
        _off, n = _deg_rows(s)

        def _fire(j, _):
            pltpu.async_copy(onesf, acc.at[idxv.at[j]], sem, add=True)
            return 0
        lax.fori_loop(0, n, _fire, 0)

        def _drain(j, _):
            pltpu.make_async_copy(dst2d_hbm.at[pl.ds(0, 1)],
                                  idxv.at[pl.ds(0, 1)], sem).wait()
            return 0
        lax.fori_loop(0, n, _drain, 0)

    @pl.when(c == 0)
    def _():
        nfull = jnp.where(s < 15, 4, 5)

        def _fire(j, _):
            pltpu.async_copy(onesi, hist.at[idxv.at[j]], sem, add=True)
            return 0
        lax.fori_loop(0, nfull, _fire, 0)

        @pl.when(s < 15)
        def _():
            pltpu.async_copy(onesi.at[pl.ds(0, 112)], hist.at[tailv], sem,
                             add=True)

        def _drain(j, _):
            pltpu.make_async_copy(dst2d_hbm.at[pl.ds(0, 1)],
                                  idxv.at[pl.ds(0, 1)], sem).wait()
            return 0
        lax.fori_loop(0, nfull, _drain, 0)

        @pl.when(s < 15)
        def _():
            pltpu.make_async_copy(ib_hbm.at[pl.ds(0, 112)], tailv, sem).wait()

    plsc.subcore_barrier()

    @pl.when(c == 1)
    def _():
        @pl.when(s < 15)
        def _():
            pltpu.sync_copy(acc.at[pl.ds(s * _RT0, _RT0)],
                            zv.at[pl.ds(0, _RT0)])
            pltpu.sync_copy(zv.at[pl.ds(0, _RT0)],
                            deg_hbm.at[pl.ds(s * _RT0, _RT0)])

        @pl.when(s == 15)
        def _():
            pltpu.sync_copy(acc.at[pl.ds(15 * _RT0, _RT15)], zv)
            pltpu.sync_copy(zv, deg_hbm.at[pl.ds(15 * _RT0, _RT15)])

    @pl.when(jnp.logical_and(c == 0, s == 0))
    def _():
        def _zp(i, _):
            stva[pl.ds(i * 16, 16)] = jnp.zeros((16,), _i32)
            stvb[pl.ds(i * 16, 16)] = jnp.zeros((16,), _i32)
            return 0
        lax.fori_loop(0, 8, _zp, 0)
        pltpu.sync_copy(hist, stva.at[pl.ds(128, 256)])

        bufs = (stva, stvb)
        for r, k in enumerate((1, 2, 4, 8, 16, 32, 64, 128)):
            srcb = bufs[r % 2]
            dstb = bufs[1 - r % 2]
            for i in range(16):
                dstb[pl.ds(128 + 16 * i, 16)] = (
                    srcb[pl.ds(128 + 16 * i, 16)]
                    + srcb[pl.ds(128 + 16 * i - k, 16)])
        for i in range(16):
            stvb[pl.ds(16 * i, 16)] = stva[pl.ds(127 + 16 * i, 16)]
        stvb[pl.ds(256, 16)] = jnp.full((16,), N, _i32)
        pltpu.sync_copy(stvb.at[pl.ds(0, 272)], starts_hbm)


_NBUF = 2


@functools.cache
def _msg_kernel_fn():
    return pl.kernel(
        _msg_body,
        out_type=jax.ShapeDtypeStruct((NC, N, D), _f32),
        mesh=_mesh(),
        scratch_types=[
            pltpu.VMEM_SHARED((N, D), _f32),
            [pltpu.VMEM((_MK, D), _f32)] * _NBUF,
            [pltpu.VMEM((_MK,), _i32)] * _NBUF,
            pltpu.VMEM((80, _MK), _i32),
            pltpu.VMEM((8, D), _f32),
            [pltpu.SemaphoreType.DMA] * _NBUF,
            [pltpu.SemaphoreType.DMA] * _NBUF,
            [pltpu.SemaphoreType.DMA] * _NBUF,
        ],
    )


def _msg_body(hs_hbm, src_hbm, dst2d_hbm, out_hbm, acc, rows, srcb, dstv,
              zbuf, sem_g, sem_s, sem_i):
    c = lax.axis_index("c")
    s = lax.axis_index("s")
    wid = c * NS + s
    off, n = _worker_rows(wid)

    @pl.when(c == 0)
    def _():
        @pl.when(s < 15)
        def _():
            pltpu.sync_copy(hs_hbm.at[pl.ds(s * _RT0, _RT0)],
                            acc.at[pl.ds(s * _RT0, _RT0)])

        @pl.when(s == 15)
        def _():
            pltpu.sync_copy(hs_hbm.at[pl.ds(15 * _RT0, _RT15)],
                            acc.at[pl.ds(15 * _RT0, _RT15)])

    @pl.when(c == 1)
    def _():
        def _zrow(i, _):
            for cc in range(8):
                zbuf[i, pl.ds(cc * 16, 16)] = jnp.zeros((16,), _f32)
            return 0
        lax.fori_loop(0, 8, _zrow, 0)
        rbase = jnp.where(s < 15, s * _RT0, 15 * _RT0)
        ncop = jnp.where(s < 15, _RT0 // 8, _RT15 // 8)

        def _zc(b, _):
            pltpu.sync_copy(zbuf, acc.at[pl.ds(rbase + b * 8, 8)])
            return 0
        lax.fori_loop(0, ncop, _zc, 0)
    _load_idx_rows(dst2d_hbm, dstv, wid, off)
    plsc.subcore_barrier()

    ebase = off * _MK

    for i in range(_NBUF):
        pltpu.sync_copy(src_hbm.at[pl.ds(ebase + i * _MK, _MK)], srcb[i])
        pltpu.async_copy(hs_hbm.at[srcb[i]], rows[i], sem_g[i])

    def _group(cg, _):
        for i in range(_NBUF):
            k = cg * _NBUF + i

            @pl.when(k < n)
            def _():
                pltpu.make_async_copy(hs_hbm.at[pl.ds(0, _MK)], rows[i],
                                      sem_g[i]).wait()

                @pl.when(k + _NBUF < n)
                def _():
                    pltpu.async_copy(
                        src_hbm.at[pl.ds(ebase + (k + _NBUF) * _MK, _MK)],
                        srcb[i], sem_i[i])
                desc = pltpu.async_copy(rows[i], acc.at[dstv.at[k]],
                                        sem_s[i], add=True)

                @pl.when(k + _NBUF < n)
                def _():
                    desc.wait()
                    pltpu.make_async_copy(
                        src_hbm.at[pl.ds(0, _MK)], srcb[i], sem_i[i]).wait()
                    pltpu.async_copy(hs_hbm.at[srcb[i]], rows[i], sem_g[i])
        return 0
    lax.fori_loop(0, 40, _group, 0)

    for i in range(_NBUF):
        pltpu.make_async_copy(hs_hbm.at[pl.ds(0, _MK)], rows[i],
                              sem_s[i]).wait()
    plsc.subcore_barrier()

    @pl.when(s < 15)
    def _():
        pltpu.sync_copy(acc.at[pl.ds(s * _RT0, _RT0)],
                        out_hbm.at[c, pl.ds(s * _RT0, _RT0)])

    @pl.when(s == 15)
    def _():
        pltpu.sync_copy(acc.at[pl.ds(15 * _RT0, _RT15)],
                        out_hbm.at[c, pl.ds(15 * _RT0, _RT15)])


_SC_C = 64


@functools.cache
def _segmax_kernel_fn():
    return pl.kernel(
        _segmax_body,
        out_type=jax.ShapeDtypeStruct((G, D), _f32),
        mesh=_mesh(),
        scratch_types=[
            pltpu.VMEM((_SC_C + 8, D), _f32),
            pltpu.VMEM((256,), _f32),
            pltpu.VMEM((16,), _i32),
            pltpu.VMEM((8, D), _f32),
        ],
    )


def _segmax_body(r_hbm, starts_hbm, st_hbm, out_hbm, rowbuf, stv, sv, obuf):
    c = lax.axis_index("c")
    s = lax.axis_index("s")
    wid = c * NS + s
    g0 = wid * 8

    pltpu.sync_copy(starts_hbm.at[pl.ds(g0, 16)], sv)
    pltpu.sync_copy(st_hbm, stv)
    svv = sv[pl.ds(0, 16)]

    ninf = jnp.full((16,), -jnp.inf, _f32)
    for k in range(8):
        gs = svv[k]
        ge = svv[k + 1]
        nch = lax.div(ge - gs + (_SC_C - 1), _SC_C)

        def _chunk(t, carry):
            base = gs + t * _SC_C
            nrows = jnp.minimum(_SC_C, ge - base)
            ab = jnp.minimum(lax.div(base, 8) * 8, N - (_SC_C + 8))
            sh = base - ab
            pltpu.sync_copy(r_hbm.at[pl.ds(ab, _SC_C + 8)], rowbuf)

            def _row(i, acc8):
                return tuple(
                    jnp.maximum(acc8[cc], rowbuf[sh + i, pl.ds(cc * 16, 16)])
                    for cc in range(8))
            return lax.fori_loop(0, nrows, _row, carry)
        acc8 = lax.fori_loop(0, nch, _chunk, (ninf,) * 8)

        for cc in range(8):
            s2 = stv[pl.ds(cc * 16, 16)]
            t2 = stv[pl.ds(D + cc * 16, 16)]
            obuf[k, pl.ds(cc * 16, 16)] = acc8[cc] * s2 + t2
    pltpu.sync_copy(obuf, out_hbm.at[pl.ds(g0, 8)])


def _k1_body(x_ref, deg_ref, w_ref, hs_ref, dinv_ref):
    dinv = lax.rsqrt(deg_ref[...] + 1.0)
    h = jnp.dot(x_ref[...], w_ref[...], preferred_element_type=_f32)
    hs_ref[...] = h * dinv
    dinv_ref[...] = dinv


def _k1(x, degcol, W):
    return pl.pallas_call(
        _k1_body,
        grid=(NBLK,),
        in_specs=[
            pl.BlockSpec((BLK, D), lambda i: (i, 0)),
            pl.BlockSpec((BLK, 1), lambda i: (i, 0)),
            pl.BlockSpec((D, D), lambda i: (0, 0)),
        ],
        out_specs=[
            pl.BlockSpec((BLK, D), lambda i: (i, 0)),
            pl.BlockSpec((BLK, 1), lambda i: (i, 0)),
        ],
        out_shape=[
            jax.ShapeDtypeStruct((N, D), _f32),
            jax.ShapeDtypeStruct((N, 1), _f32),
        ],
    )(x, degcol, W)


def _k23_body(p0_ref, p1_ref, dinv_ref, b_ref, g_ref, be_ref, w_ref,
              hs2_ref, rbuf, ssum, ssq, stscr):
    t = pl.program_id(0)
    i = pl.program_id(1)

    @pl.when(t == 0)
    def _():
        v = (dinv_ref[...] * (p0_ref[...] + p1_ref[...])
             + b_ref[...])
        r = jnp.maximum(v, 0.0)
        rbuf[pl.ds(i * BLK, BLK), :] = r
        cs = jnp.sum(r, axis=0, keepdims=True)
        cq = jnp.sum(r * r, axis=0, keepdims=True)

        @pl.when(i == 0)
        def _():
            ssum[...] = cs
            ssq[...] = cq

        @pl.when(i > 0)
        def _():
            ssum[...] += cs
            ssq[...] += cq

        @pl.when(i == NBLK - 1)
        def _():
            mean = ssum[...] / N
            var = ssq[...] / N - mean * mean
            sc = g_ref[...] * lax.rsqrt(var + EPS)
            stscr[...] = jnp.concatenate(
                [sc, be_ref[...] - mean * sc], axis=0)

    @pl.when(t == 1)
    def _():
        sc = stscr[0:1, :]
        sh = stscr[1:2, :]
        x2 = rbuf[pl.ds(i * BLK, BLK), :] * sc + sh
        h2 = jnp.dot(x2, w_ref[...], preferred_element_type=_f32)
        hs2_ref[...] = h2 * dinv_ref[...]


def _k23(p0, p1, dinvcol, b, g, be, W2):
    blk_p0 = pl.BlockSpec((BLK, D), lambda t, i: (i * (1 - t), 0))
    return pl.pallas_call(
        _k23_body,
        grid=(2, NBLK),
        in_specs=[
            blk_p0,
            blk_p0,
            pl.BlockSpec((BLK, 1), lambda t, i: (i, 0)),
            pl.BlockSpec((1, D), lambda t, i: (0, 0)),
            pl.BlockSpec((1, D), lambda t, i: (0, 0)),
            pl.BlockSpec((1, D), lambda t, i: (0, 0)),
            pl.BlockSpec((D, D), lambda t, i: (0, 0)),
        ],
        out_specs=pl.BlockSpec((BLK, D), lambda t, i: (i * t, 0)),
        out_shape=jax.ShapeDtypeStruct((N, D), _f32),
        scratch_shapes=[
            pltpu.VMEM((N, D), _f32),
            pltpu.VMEM((1, D), _f32),
            pltpu.VMEM((1, D), _f32),
            pltpu.VMEM((2, D), _f32),
        ],
    )(p0, p1, dinvcol, b, g, be, W2)


def _k2_body(p0_ref, p1_ref, dinv_ref, b_ref, g_ref, be_ref,
             r_ref, st_ref, ssum, ssq):
    i = pl.program_id(0)
    v = dinv_ref[...] * (p0_ref[...] + p1_ref[...]) + b_ref[...]
    r = jnp.maximum(v, 0.0)
    r_ref[...] = r
    cs = jnp.sum(r, axis=0, keepdims=True)
    cq = jnp.sum(r * r, axis=0, keepdims=True)

    @pl.when(i == 0)
    def _():
        ssum[...] = cs
        ssq[...] = cq

    @pl.when(i > 0)
    def _():
        ssum[...] += cs
        ssq[...] += cq

    @pl.when(i == NBLK - 1)
    def _():
        mean = ssum[...] / N
        var = ssq[...] / N - mean * mean
        sc = g_ref[...] * lax.rsqrt(var + EPS)
        st_ref[...] = jnp.concatenate([sc, be_ref[...] - mean * sc], axis=0)


def _k2(p0, p1, dinvcol, b, g, be):
    return pl.pallas_call(
        _k2_body,
        grid=(NBLK,),
        in_specs=[
            pl.BlockSpec((BLK, D), lambda i: (i, 0)),
            pl.BlockSpec((BLK, D), lambda i: (i, 0)),
            pl.BlockSpec((BLK, 1), lambda i: (i, 0)),
            pl.BlockSpec((1, D), lambda i: (0, 0)),
            pl.BlockSpec((1, D), lambda i: (0, 0)),
            pl.BlockSpec((1, D), lambda i: (0, 0)),
        ],
        out_specs=[
            pl.BlockSpec((BLK, D), lambda i: (i, 0)),
            pl.BlockSpec((2, D), lambda i: (0, 0)),
        ],
        out_shape=[
            jax.ShapeDtypeStruct((N, D), _f32),
            jax.ShapeDtypeStruct((2, D), _f32),
        ],
        scratch_shapes=[
            pltpu.VMEM((1, D), _f32),
            pltpu.VMEM((1, D), _f32),
        ],
    )(p0, p1, dinvcol, b, g, be)


def kernel(input_feature, input_adj, ibatch, W1, b1, g1, be1, W2, b2, g2, be2):
    src1d = input_adj[0]
    dst2d = input_adj[1].reshape(ER, 128)

    deg, starts = _deg_kernel_fn()(dst2d, ibatch)
    hs1, dinvcol = _k1(input_feature, deg[:, None], W1)

    parts1 = _msg_kernel_fn()(hs1, src1d, dst2d)
    hs2 = _k23(parts1[0], parts1[1], dinvcol,
               b1[None, :], g1[None, :], be1[None, :], W2)

    parts2 = _msg_kernel_fn()(hs2, src1d, dst2d)
    r2, st2 = _k2(parts2[0], parts2[1], dinvcol,
                  b2[None, :], g2[None, :], be2[None, :])

    out = _segmax_kernel_fn()(r2, starts, st2.reshape(2 * D))
    return out


# speedup vs baseline: 29.1982x; 1.0594x over previous
import functools

import jax
import jax.numpy as jnp
from jax import lax
from jax.experimental import pallas as pl
from jax.experimental.pallas import tpu as pltpu
from jax.experimental.pallas import tpu_sc as plsc

N = 10000
NR = 10080
E = 320000
D = 128
G = 256
EPS = 1e-5

NC, NS = 2, 16

BLK = 400
NBLK = N // BLK

_f32 = jnp.float32
_i32 = jnp.int32

@functools.cache
def _mesh():
    return plsc.VectorSubcoreMesh(core_axis_name="c", subcore_axis_name="s",
                                  num_cores=NC, num_subcores=NS)


ER = E // 128
_MK = 128


def _worker_rows(w):
    off = jnp.where(w < 24, 80 * w, 1920 + 72 * (w - 24))
    n = jnp.where(w < 24, 80, jnp.where(w == 31, 76, 72))
    return off, n


def _load_idx_rows(src2d_hbm, buf, w, off):
    pltpu.sync_copy(src2d_hbm.at[pl.ds(off, 72)], buf.at[pl.ds(0, 72)])

    @pl.when(w < 24)
    def _():
        pltpu.sync_copy(src2d_hbm.at[pl.ds(off + 72, 8)], buf.at[pl.ds(72, 8)])

    @pl.when(w == 31)
    def _():
        pltpu.sync_copy(src2d_hbm.at[pl.ds(off + 72, 4)], buf.at[pl.ds(72, 4)])


_RT0 = 624
_RT15 = N - 15 * _RT0


def _deg_rows(t):
    off = jnp.where(t < 8, 160 * t, 1280 + 152 * (t - 8))
    n = jnp.where(t < 8, 160, jnp.where(t == 15, 156, 152))
    return off, n


@functools.cache
def _deg_kernel_fn():
    return pl.kernel(
        _deg_body,
        out_type=(jax.ShapeDtypeStruct((N,), _f32),
                  jax.ShapeDtypeStruct((272,), _i32)),
        mesh=_mesh(),
        scratch_types=[
            pltpu.VMEM_SHARED((N,), _f32),
            pltpu.VMEM_SHARED((256,), _i32),
            pltpu.VMEM((160, _MK), _i32),
            pltpu.VMEM((_MK,), _f32),
            pltpu.VMEM((_MK,), _i32),
            pltpu.VMEM((112,), _i32),
            pltpu.VMEM((_RT15,), _f32),
            pltpu.VMEM((384,), _i32),
            pltpu.VMEM((384,), _i32),
            pltpu.SemaphoreType.DMA,
        ],
    )


def _deg_body(dst2d_hbm, ib_hbm, deg_hbm, starts_hbm, acc, hist, idxv,
              onesf, onesi, tailv, zv, stva, stvb, sem):
    c = lax.axis_index("c")
    s = lax.axis_index("s")

    def _zs(i, _):
        zv[pl.ds(i * 16, 16)] = jnp.zeros((16,), _f32)
        return 0
    lax.fori_loop(0, _RT15 // 16, _zs, 0)

    def _os(i, _):
        onesf[pl.ds(i * 16, 16)] = jnp.ones((16,), _f32)
        onesi[pl.ds(i * 16, 16)] = jnp.ones((16,), _i32)
        return 0
    lax.fori_loop(0, _MK // 16, _os, 0)

    @pl.when(c == 1)
    def _():
        @pl.when(s < 15)
        def _():
            pltpu.sync_copy(zv.at[pl.ds(0, _RT0)],
                            acc.at[pl.ds(s * _RT0, _RT0)])

        @pl.when(s == 15)
        def _():
            pltpu.sync_copy(zv, acc.at[pl.ds(15 * _RT0, _RT15)])

        off, _n = _deg_rows(s)
        pltpu.sync_copy(dst2d_hbm.at[pl.ds(off, 152)],
                        idxv.at[pl.ds(0, 152)])

        @pl.when(s < 8)
        def _():
            pltpu.sync_copy(dst2d_hbm.at[pl.ds(off + 152, 8)],
                            idxv.at[pl.ds(152, 8)])

        @pl.when(s == 15)
        def _():
            pltpu.sync_copy(dst2d_hbm.at[pl.ds(off + 152, 4)],
                            idxv.at[pl.ds(152, 4)])

    @pl.when(c == 0)
    def _():
        @pl.when(s == 0)
        def _():
            def _zh(i, _):
                stva[pl.ds(i * 16, 16)] = jnp.zeros((16,), _i32)
                return 0
            lax.fori_loop(0, 16, _zh, 0)
            pltpu.sync_copy(stva.at[pl.ds(0, 256)], hist)

        ibase = jnp.where(s < 15, s * _RT0, 15 * _RT0)
        nfull = jnp.where(s < 15, 4, 5)

        def _ldc(j, _):
            pltpu.sync_copy(ib_hbm.at[pl.ds(ibase + j * _MK, _MK)],
                            idxv.at[j])
            return 0
        lax.fori_loop(0, nfull, _ldc, 0)

        @pl.when(s < 15)
        def _():
            pltpu.sync_copy(ib_hbm.at[pl.ds(ibase + 4 * _MK, 112)], tailv)

    plsc.subcore_barrier()

    @pl.when(c == 1)
    def _():
        _off, n = _deg_rows(s)

        def _fire(j, _):
            pltpu.async_copy(onesf, acc.at[idxv.at[j]], sem, add=True)
            return 0
        lax.fori_loop(0, n, _fire, 0)

        def _drain(j, _):
            pltpu.make_async_copy(dst2d_hbm.at[pl.ds(0, 1)],
                                  idxv.at[pl.ds(0, 1)], sem).wait()
            return 0
        lax.fori_loop(0, n, _drain, 0)

    @pl.when(c == 0)
    def _():
        nfull = jnp.where(s < 15, 4, 5)

        def _fire(j, _):
            pltpu.async_copy(onesi, hist.at[idxv.at[j]], sem, add=True)
            return 0
        lax.fori_loop(0, nfull, _fire, 0)

        @pl.when(s < 15)
        def _():
            pltpu.async_copy(onesi.at[pl.ds(0, 112)], hist.at[tailv], sem,
                             add=True)

        def _drain(j, _):
            pltpu.make_async_copy(dst2d_hbm.at[pl.ds(0, 1)],
                                  idxv.at[pl.ds(0, 1)], sem).wait()
            return 0
        lax.fori_loop(0, nfull, _drain, 0)

        @pl.when(s < 15)
        def _():
            pltpu.make_async_copy(ib_hbm.at[pl.ds(0, 112)], tailv, sem).wait()

    plsc.subcore_barrier()

    @pl.when(c == 1)
    def _():
        @pl.when(s < 15)
        def _():
            pltpu.sync_copy(acc.at[pl.ds(s * _RT0, _RT0)],
                            zv.at[pl.ds(0, _RT0)])
            pltpu.sync_copy(zv.at[pl.ds(0, _RT0)],
                            deg_hbm.at[pl.ds(s * _RT0, _RT0)])

        @pl.when(s == 15)
        def _():
            pltpu.sync_copy(acc.at[pl.ds(15 * _RT0, _RT15)], zv)
            pltpu.sync_copy(zv, deg_hbm.at[pl.ds(15 * _RT0, _RT15)])

    @pl.when(jnp.logical_and(c == 0, s == 0))
    def _():
        def _zp(i, _):
            stva[pl.ds(i * 16, 16)] = jnp.zeros((16,), _i32)
            stvb[pl.ds(i * 16, 16)] = jnp.zeros((16,), _i32)
            return 0
        lax.fori_loop(0, 8, _zp, 0)
        pltpu.sync_copy(hist, stva.at[pl.ds(128, 256)])

        bufs = (stva, stvb)
        for r, k in enumerate((1, 2, 4, 8, 16, 32, 64, 128)):
            srcb = bufs[r % 2]
            dstb = bufs[1 - r % 2]
            for i in range(16):
                dstb[pl.ds(128 + 16 * i, 16)] = (
                    srcb[pl.ds(128 + 16 * i, 16)]
                    + srcb[pl.ds(128 + 16 * i - k, 16)])
        for i in range(16):
            stvb[pl.ds(16 * i, 16)] = stva[pl.ds(127 + 16 * i, 16)]
        stvb[pl.ds(256, 16)] = jnp.full((16,), N, _i32)
        pltpu.sync_copy(stvb.at[pl.ds(0, 272)], starts_hbm)


_NBUF = 3


@functools.cache
def _msg_kernel_fn():
    return pl.kernel(
        _msg_body,
        out_type=jax.ShapeDtypeStruct((NC, N, D), _f32),
        mesh=_mesh(),
        scratch_types=[
            pltpu.VMEM_SHARED((N, D), _f32),
            [pltpu.VMEM((_MK, D), _f32)] * _NBUF,
            [pltpu.VMEM((_MK,), _i32)] * _NBUF,
            [pltpu.VMEM((_MK,), _i32)] * _NBUF,
            pltpu.VMEM((8, D), _f32),
            [pltpu.SemaphoreType.DMA] * _NBUF,
            [pltpu.SemaphoreType.DMA] * _NBUF,
            [pltpu.SemaphoreType.DMA] * _NBUF,
            [pltpu.SemaphoreType.DMA] * _NBUF,
        ],
    )


def _msg_body(hs_hbm, src_hbm, dst_hbm, out_hbm, acc, rows, srcb, dstb,
              zbuf, sem_g, sem_s, sem_i, sem_j):
    c = lax.axis_index("c")
    s = lax.axis_index("s")
    wid = c * NS + s
    off, n = _worker_rows(wid)

    @pl.when(c == 0)
    def _():
        @pl.when(s < 15)
        def _():
            pltpu.sync_copy(hs_hbm.at[pl.ds(s * _RT0, _RT0)],
                            acc.at[pl.ds(s * _RT0, _RT0)])

        @pl.when(s == 15)
        def _():
            pltpu.sync_copy(hs_hbm.at[pl.ds(15 * _RT0, _RT15)],
                            acc.at[pl.ds(15 * _RT0, _RT15)])

    @pl.when(c == 1)
    def _():
        def _zrow(i, _):
            for cc in range(8):
                zbuf[i, pl.ds(cc * 16, 16)] = jnp.zeros((16,), _f32)
            return 0
        lax.fori_loop(0, 8, _zrow, 0)
        rbase = jnp.where(s < 15, s * _RT0, 15 * _RT0)
        ncop = jnp.where(s < 15, _RT0 // 8, _RT15 // 8)

        def _zc(b, _):
            pltpu.sync_copy(zbuf, acc.at[pl.ds(rbase + b * 8, 8)])
            return 0
        lax.fori_loop(0, ncop, _zc, 0)
    plsc.subcore_barrier()

    ebase = off * _MK

    for i in range(_NBUF):
        pltpu.sync_copy(src_hbm.at[pl.ds(ebase + i * _MK, _MK)], srcb[i])
        pltpu.sync_copy(dst_hbm.at[pl.ds(ebase + i * _MK, _MK)], dstb[i])
        pltpu.async_copy(hs_hbm.at[srcb[i]], rows[i], sem_g[i])

    def _group(cg, _):
        for i in range(_NBUF):
            k = cg * _NBUF + i

            @pl.when(k < n)
            def _():
                pltpu.make_async_copy(hs_hbm.at[pl.ds(0, _MK)], rows[i],
                                      sem_g[i]).wait()

                @pl.when(k + _NBUF < n)
                def _():
                    pltpu.async_copy(
                        src_hbm.at[pl.ds(ebase + (k + _NBUF) * _MK, _MK)],
                        srcb[i], sem_i[i])

                @pl.when(k >= _NBUF)
                def _():
                    pltpu.make_async_copy(
                        dst_hbm.at[pl.ds(0, _MK)], dstb[i], sem_j[i]).wait()
                desc = pltpu.async_copy(rows[i], acc.at[dstb[i]],
                                        sem_s[i], add=True)

                @pl.when(k + _NBUF < n)
                def _():
                    desc.wait()
                    pltpu.make_async_copy(
                        src_hbm.at[pl.ds(0, _MK)], srcb[i], sem_i[i]).wait()
                    pltpu.async_copy(
                        dst_hbm.at[pl.ds(ebase + (k + _NBUF) * _MK, _MK)],
                        dstb[i], sem_j[i])
                    pltpu.async_copy(hs_hbm.at[srcb[i]], rows[i], sem_g[i])
        return 0
    lax.fori_loop(0, 27, _group, 0)

    for i in range(_NBUF):
        pltpu.make_async_copy(hs_hbm.at[pl.ds(0, _MK)], rows[i],
                              sem_s[i]).wait()
    plsc.subcore_barrier()

    @pl.when(s < 15)
    def _():
        pltpu.sync_copy(acc.at[pl.ds(s * _RT0, _RT0)],
                        out_hbm.at[c, pl.ds(s * _RT0, _RT0)])

    @pl.when(s == 15)
    def _():
        pltpu.sync_copy(acc.at[pl.ds(15 * _RT0, _RT15)],
                        out_hbm.at[c, pl.ds(15 * _RT0, _RT15)])


_SC_C = 64


@functools.cache
def _segmax_kernel_fn():
    return pl.kernel(
        _segmax_body,
        out_type=jax.ShapeDtypeStruct((G, D), _f32),
        mesh=_mesh(),
        scratch_types=[
            pltpu.VMEM((_SC_C + 8, D), _f32),
            pltpu.VMEM((256,), _f32),
            pltpu.VMEM((16,), _i32),
            pltpu.VMEM((8, D), _f32),
        ],
    )


def _segmax_body(r_hbm, starts_hbm, st_hbm, out_hbm, rowbuf, stv, sv, obuf):
    c = lax.axis_index("c")
    s = lax.axis_index("s")
    wid = c * NS + s
    g0 = wid * 8

    pltpu.sync_copy(starts_hbm.at[pl.ds(g0, 16)], sv)
    pltpu.sync_copy(st_hbm, stv)
    svv = sv[pl.ds(0, 16)]

    ninf = jnp.full((16,), -jnp.inf, _f32)
    for k in range(8):
        gs = svv[k]
        ge = svv[k + 1]
        nch = lax.div(ge - gs + (_SC_C - 1), _SC_C)

        def _chunk(t, carry):
            base = gs + t * _SC_C
            nrows = jnp.minimum(_SC_C, ge - base)
            ab = jnp.minimum(lax.div(base, 8) * 8, N - (_SC_C + 8))
            sh = base - ab
            pltpu.sync_copy(r_hbm.at[pl.ds(ab, _SC_C + 8)], rowbuf)

            def _row(i, acc8):
                return tuple(
                    jnp.maximum(acc8[cc], rowbuf[sh + i, pl.ds(cc * 16, 16)])
                    for cc in range(8))
            return lax.fori_loop(0, nrows, _row, carry)
        acc8 = lax.fori_loop(0, nch, _chunk, (ninf,) * 8)

        for cc in range(8):
            s2 = stv[pl.ds(cc * 16, 16)]
            t2 = stv[pl.ds(D + cc * 16, 16)]
            obuf[k, pl.ds(cc * 16, 16)] = acc8[cc] * s2 + t2
    pltpu.sync_copy(obuf, out_hbm.at[pl.ds(g0, 8)])


def _k1_body(x_ref, deg_ref, w_ref, hs_ref, dinv_ref):
    dinv = lax.rsqrt(deg_ref[...] + 1.0)
    h = jnp.dot(x_ref[...], w_ref[...], preferred_element_type=_f32)
    hs_ref[...] = h * dinv
    dinv_ref[...] = dinv


def _k1(x, degcol, W):
    return pl.pallas_call(
        _k1_body,
        grid=(NBLK,),
        in_specs=[
            pl.BlockSpec((BLK, D), lambda i: (i, 0)),
            pl.BlockSpec((BLK, 1), lambda i: (i, 0)),
            pl.BlockSpec((D, D), lambda i: (0, 0)),
        ],
        out_specs=[
            pl.BlockSpec((BLK, D), lambda i: (i, 0)),
            pl.BlockSpec((BLK, 1), lambda i: (i, 0)),
        ],
        out_shape=[
            jax.ShapeDtypeStruct((N, D), _f32),
            jax.ShapeDtypeStruct((N, 1), _f32),
        ],
    )(x, degcol, W)


def _k23_body(p0_ref, p1_ref, dinv_ref, b_ref, g_ref, be_ref, w_ref,
              hs2_ref, rbuf, ssum, ssq, stscr):
    t = pl.program_id(0)
    i = pl.program_id(1)

    @pl.when(t == 0)
    def _():
        v = (dinv_ref[...] * (p0_ref[...] + p1_ref[...])
             + b_ref[...])
        r = jnp.maximum(v, 0.0)
        rbuf[pl.ds(i * BLK, BLK), :] = r
        cs = jnp.sum(r, axis=0, keepdims=True)
        cq = jnp.sum(r * r, axis=0, keepdims=True)

        @pl.when(i == 0)
        def _():
            ssum[...] = cs
            ssq[...] = cq

        @pl.when(i > 0)
        def _():
            ssum[...] += cs
            ssq[...] += cq

        @pl.when(i == NBLK - 1)
        def _():
            mean = ssum[...] / N
            var = ssq[...] / N - mean * mean
            sc = g_ref[...] * lax.rsqrt(var + EPS)
            stscr[...] = jnp.concatenate(
                [sc, be_ref[...] - mean * sc], axis=0)

    @pl.when(t == 1)
    def _():
        sc = stscr[0:1, :]
        sh = stscr[1:2, :]
        x2 = rbuf[pl.ds(i * BLK, BLK), :] * sc + sh
        h2 = jnp.dot(x2, w_ref[...], preferred_element_type=_f32)
        hs2_ref[...] = h2 * dinv_ref[...]


def _k23(p0, p1, dinvcol, b, g, be, W2):
    blk_p0 = pl.BlockSpec((BLK, D), lambda t, i: (i * (1 - t), 0))
    return pl.pallas_call(
        _k23_body,
        grid=(2, NBLK),
        in_specs=[
            blk_p0,
            blk_p0,
            pl.BlockSpec((BLK, 1), lambda t, i: (i, 0)),
            pl.BlockSpec((1, D), lambda t, i: (0, 0)),
            pl.BlockSpec((1, D), lambda t, i: (0, 0)),
            pl.BlockSpec((1, D), lambda t, i: (0, 0)),
            pl.BlockSpec((D, D), lambda t, i: (0, 0)),
        ],
        out_specs=pl.BlockSpec((BLK, D), lambda t, i: (i * t, 0)),
        out_shape=jax.ShapeDtypeStruct((N, D), _f32),
        scratch_shapes=[
            pltpu.VMEM((N, D), _f32),
            pltpu.VMEM((1, D), _f32),
            pltpu.VMEM((1, D), _f32),
            pltpu.VMEM((2, D), _f32),
        ],
    )(p0, p1, dinvcol, b, g, be, W2)


def _k2_body(p0_ref, p1_ref, dinv_ref, b_ref, g_ref, be_ref,
             r_ref, st_ref, ssum, ssq):
    i = pl.program_id(0)
    v = dinv_ref[...] * (p0_ref[...] + p1_ref[...]) + b_ref[...]
    r = jnp.maximum(v, 0.0)
    r_ref[...] = r
    cs = jnp.sum(r, axis=0, keepdims=True)
    cq = jnp.sum(r * r, axis=0, keepdims=True)

    @pl.when(i == 0)
    def _():
        ssum[...] = cs
        ssq[...] = cq

    @pl.when(i > 0)
    def _():
        ssum[...] += cs
        ssq[...] += cq

    @pl.when(i == NBLK - 1)
    def _():
        mean = ssum[...] / N
        var = ssq[...] / N - mean * mean
        sc = g_ref[...] * lax.rsqrt(var + EPS)
        st_ref[...] = jnp.concatenate([sc, be_ref[...] - mean * sc], axis=0)


def _k2(p0, p1, dinvcol, b, g, be):
    return pl.pallas_call(
        _k2_body,
        grid=(NBLK,),
        in_specs=[
            pl.BlockSpec((BLK, D), lambda i: (i, 0)),
            pl.BlockSpec((BLK, D), lambda i: (i, 0)),
            pl.BlockSpec((BLK, 1), lambda i: (i, 0)),
            pl.BlockSpec((1, D), lambda i: (0, 0)),
            pl.BlockSpec((1, D), lambda i: (0, 0)),
            pl.BlockSpec((1, D), lambda i: (0, 0)),
        ],
        out_specs=[
            pl.BlockSpec((BLK, D), lambda i: (i, 0)),
            pl.BlockSpec((2, D), lambda i: (0, 0)),
        ],
        out_shape=[
            jax.ShapeDtypeStruct((N, D), _f32),
            jax.ShapeDtypeStruct((2, D), _f32),
        ],
        scratch_shapes=[
            pltpu.VMEM((1, D), _f32),
            pltpu.VMEM((1, D), _f32),
        ],
    )(p0, p1, dinvcol, b, g, be)


def kernel(input_feature, input_adj, ibatch, W1, b1, g1, be1, W2, b2, g2, be2):
    src1d = input_adj[0]
    dst2d = input_adj[1].reshape(ER, 128)

    deg, starts = _deg_kernel_fn()(dst2d, ibatch)
    hs1, dinvcol = _k1(input_feature, deg[:, None], W1)

    dst1d = input_adj[1]
    parts1 = _msg_kernel_fn()(hs1, src1d, dst1d)
    hs2 = _k23(parts1[0], parts1[1], dinvcol,
               b1[None, :], g1[None, :], be1[None, :], W2)

    parts2 = _msg_kernel_fn()(hs2, src1d, dst1d)
    r2, st2 = _k2(parts2[0], parts2[1], dinvcol,
                  b2[None, :], g2[None, :], be2[None, :])

    out = _segmax_kernel_fn()(r2, starts, st2.reshape(2 * D))
    return out


# speedup vs baseline: 29.2373x; 1.0013x over previous
import functools

import jax
import jax.numpy as jnp
from jax import lax
from jax.experimental import pallas as pl
from jax.experimental.pallas import tpu as pltpu
from jax.experimental.pallas import tpu_sc as plsc

N = 10000
E = 320000
D = 128
G = 256
EPS = 1e-5

NC, NS = 2, 16

BLK = 400
NBLK = N // BLK

_f32 = jnp.float32
_i32 = jnp.int32

@functools.cache
def _mesh():
    return plsc.VectorSubcoreMesh(core_axis_name="c", subcore_axis_name="s",
                                  num_cores=NC, num_subcores=NS)


ER = E // 128
_MK = 128


def _worker_rows(w):
    off = jnp.where(w < 24, 80 * w, 1920 + 72 * (w - 24))
    n = jnp.where(w < 24, 80, jnp.where(w == 31, 76, 72))
    return off, n


_RT0 = 624
_RT15 = N - 15 * _RT0


def _deg_rows(t):
    off = jnp.where(t < 8, 160 * t, 1280 + 152 * (t - 8))
    n = jnp.where(t < 8, 160, jnp.where(t == 15, 156, 152))
    return off, n


@functools.cache
def _deg_kernel_fn():
    return pl.kernel(
        _deg_body,
        out_type=(jax.ShapeDtypeStruct((N,), _f32),
                  jax.ShapeDtypeStruct((272,), _i32)),
        mesh=_mesh(),
        scratch_types=[
            pltpu.VMEM_SHARED((N,), _f32),
            pltpu.VMEM_SHARED((256,), _i32),
            pltpu.VMEM((160, _MK), _i32),
            pltpu.VMEM((_MK,), _f32),
            pltpu.VMEM((_MK,), _i32),
            pltpu.VMEM((112,), _i32),
            pltpu.VMEM((_RT15,), _f32),
            pltpu.VMEM((384,), _i32),
            pltpu.VMEM((384,), _i32),
            pltpu.SemaphoreType.DMA,
        ],
    )


def _deg_body(dst2d_hbm, ib_hbm, deg_hbm, starts_hbm, acc, hist, idxv,
              onesf, onesi, tailv, zv, stva, stvb, sem):
    c = lax.axis_index("c")
    s = lax.axis_index("s")

    def _zs(i, _):
        zv[pl.ds(i * 16, 16)] = jnp.zeros((16,), _f32)
        return 0
    lax.fori_loop(0, _RT15 // 16, _zs, 0)

    def _os(i, _):
        onesf[pl.ds(i * 16, 16)] = jnp.ones((16,), _f32)
        onesi[pl.ds(i * 16, 16)] = jnp.ones((16,), _i32)
        return 0
    lax.fori_loop(0, _MK // 16, _os, 0)

    @pl.when(c == 1)
    def _():
        @pl.when(s < 15)
        def _():
            pltpu.sync_copy(zv.at[pl.ds(0, _RT0)],
                            acc.at[pl.ds(s * _RT0, _RT0)])

        @pl.when(s == 15)
        def _():
            pltpu.sync_copy(zv, acc.at[pl.ds(15 * _RT0, _RT15)])

        off, _n = _deg_rows(s)
        pltpu.sync_copy(dst2d_hbm.at[pl.ds(off, 152)],
                        idxv.at[pl.ds(0, 152)])

        @pl.when(s < 8)
        def _():
            pltpu.sync_copy(dst2d_hbm.at[pl.ds(off + 152, 8)],
                            idxv.at[pl.ds(152, 8)])

        @pl.when(s == 15)
        def _():
            pltpu.sync_copy(dst2d_hbm.at[pl.ds(off + 152, 4)],
                            idxv.at[pl.ds(152, 4)])

    @pl.when(c == 0)
    def _():
        @pl.when(s == 0)
        def _():
            def _zh(i, _):
                stva[pl.ds(i * 16, 16)] = jnp.zeros((16,), _i32)
                return 0
            lax.fori_loop(0, 16, _zh, 0)
            pltpu.sync_copy(stva.at[pl.ds(0, 256)], hist)

        ibase = jnp.where(s < 15, s * _RT0, 15 * _RT0)
        nfull = jnp.where(s < 15, 4, 5)

        def _ldc(j, _):
            pltpu.sync_copy(ib_hbm.at[pl.ds(ibase + j * _MK, _MK)],
                            idxv.at[j])
            return 0
        lax.fori_loop(0, nfull, _ldc, 0)

        @pl.when(s < 15)
        def _():
            pltpu.sync_copy(ib_hbm.at[pl.ds(ibase + 4 * _MK, 112)], tailv)

    plsc.subcore_barrier()

    @pl.when(c == 1)
    def _():
        _off, n = _deg_rows(s)

        def _fire(j, _):
            pltpu.async_copy(onesf, acc.at[idxv.at[j]], sem, add=True)
            return 0
        lax.fori_loop(0, n, _fire, 0)

        def _drain(j, _):
            pltpu.make_async_copy(dst2d_hbm.at[pl.ds(0, 1)],
                                  idxv.at[pl.ds(0, 1)], sem).wait()
            return 0
        lax.fori_loop(0, n, _drain, 0)

    @pl.when(c == 0)
    def _():
        nfull = jnp.where(s < 15, 4, 5)

        def _fire(j, _):
            pltpu.async_copy(onesi, hist.at[idxv.at[j]], sem, add=True)
            return 0
        lax.fori_loop(0, nfull, _fire, 0)

        @pl.when(s < 15)
        def _():
            pltpu.async_copy(onesi.at[pl.ds(0, 112)], hist.at[tailv], sem,
                             add=True)

        def _drain(j, _):
            pltpu.make_async_copy(dst2d_hbm.at[pl.ds(0, 1)],
                                  idxv.at[pl.ds(0, 1)], sem).wait()
            return 0
        lax.fori_loop(0, nfull, _drain, 0)

        @pl.when(s < 15)
        def _():
            pltpu.make_async_copy(ib_hbm.at[pl.ds(0, 112)], tailv, sem).wait()

    plsc.subcore_barrier()

    @pl.when(c == 1)
    def _():
        @pl.when(s < 15)
        def _():
            pltpu.sync_copy(acc.at[pl.ds(s * _RT0, _RT0)],
                            zv.at[pl.ds(0, _RT0)])
            pltpu.sync_copy(zv.at[pl.ds(0, _RT0)],
                            deg_hbm.at[pl.ds(s * _RT0, _RT0)])

        @pl.when(s == 15)
        def _():
            pltpu.sync_copy(acc.at[pl.ds(15 * _RT0, _RT15)], zv)
            pltpu.sync_copy(zv, deg_hbm.at[pl.ds(15 * _RT0, _RT15)])

    @pl.when(jnp.logical_and(c == 0, s == 0))
    def _():
        def _zp(i, _):
            stva[pl.ds(i * 16, 16)] = jnp.zeros((16,), _i32)
            stvb[pl.ds(i * 16, 16)] = jnp.zeros((16,), _i32)
            return 0
        lax.fori_loop(0, 8, _zp, 0)
        pltpu.sync_copy(hist, stva.at[pl.ds(128, 256)])

        bufs = (stva, stvb)
        for r, k in enumerate((1, 2, 4, 8, 16, 32, 64, 128)):
            srcb = bufs[r % 2]
            dstb = bufs[1 - r % 2]
            for i in range(16):
                dstb[pl.ds(128 + 16 * i, 16)] = (
                    srcb[pl.ds(128 + 16 * i, 16)]
                    + srcb[pl.ds(128 + 16 * i - k, 16)])
        for i in range(16):
            stvb[pl.ds(16 * i, 16)] = stva[pl.ds(127 + 16 * i, 16)]
        stvb[pl.ds(256, 16)] = jnp.full((16,), N, _i32)
        pltpu.sync_copy(stvb.at[pl.ds(0, 272)], starts_hbm)


_NBUF = 3


@functools.cache
def _msg_kernel_fn():
    return pl.kernel(
        _msg_body,
        out_type=jax.ShapeDtypeStruct((NC, N, D), _f32),
        mesh=_mesh(),
        scratch_types=[
            pltpu.VMEM_SHARED((N, D), _f32),
            [pltpu.VMEM((_MK, D), _f32)] * _NBUF,
            [pltpu.VMEM((_MK,), _i32)] * _NBUF,
            [pltpu.VMEM((_MK,), _i32)] * _NBUF,
            pltpu.VMEM((8, D), _f32),
            [pltpu.SemaphoreType.DMA] * _NBUF,
            [pltpu.SemaphoreType.DMA] * _NBUF,
            [pltpu.SemaphoreType.DMA] * _NBUF,
            [pltpu.SemaphoreType.DMA] * _NBUF,
        ],
    )


def _msg_body(hs_hbm, src_hbm, dst_hbm, out_hbm, acc, rows, srcb, dstb,
              zbuf, sem_g, sem_s, sem_i, sem_j):
    c = lax.axis_index("c")
    s = lax.axis_index("s")
    wid = c * NS + s
    off, n = _worker_rows(wid)

    @pl.when(c == 0)
    def _():
        @pl.when(s < 15)
        def _():
            pltpu.sync_copy(hs_hbm.at[pl.ds(s * _RT0, _RT0)],
                            acc.at[pl.ds(s * _RT0, _RT0)])

        @pl.when(s == 15)
        def _():
            pltpu.sync_copy(hs_hbm.at[pl.ds(15 * _RT0, _RT15)],
                            acc.at[pl.ds(15 * _RT0, _RT15)])

    @pl.when(c == 1)
    def _():
        def _zrow(i, _):
            for cc in range(8):
                zbuf[i, pl.ds(cc * 16, 16)] = jnp.zeros((16,), _f32)
            return 0
        lax.fori_loop(0, 8, _zrow, 0)
        rbase = jnp.where(s < 15, s * _RT0, 15 * _RT0)
        ncop = jnp.where(s < 15, _RT0 // 8, _RT15 // 8)

        def _zc(b, _):
            pltpu.sync_copy(zbuf, acc.at[pl.ds(rbase + b * 8, 8)])
            return 0
        lax.fori_loop(0, ncop, _zc, 0)
    plsc.subcore_barrier()

    ebase = off * _MK

    for i in range(_NBUF):
        pltpu.sync_copy(src_hbm.at[pl.ds(ebase + i * _MK, _MK)], srcb[i])
        pltpu.sync_copy(dst_hbm.at[pl.ds(ebase + i * _MK, _MK)], dstb[i])
        pltpu.async_copy(hs_hbm.at[srcb[i]], rows[i], sem_g[i])

    def _group(cg, _):
        for i in range(_NBUF):
            k = cg * _NBUF + i

            @pl.when(k < n)
            def _():
                pltpu.make_async_copy(hs_hbm.at[pl.ds(0, _MK)], rows[i],
                                      sem_g[i]).wait()

                @pl.when(k + _NBUF < n)
                def _():
                    pltpu.async_copy(
                        src_hbm.at[pl.ds(ebase + (k + _NBUF) * _MK, _MK)],
                        srcb[i], sem_i[i])

                @pl.when(k >= _NBUF)
                def _():
                    pltpu.make_async_copy(
                        dst_hbm.at[pl.ds(0, _MK)], dstb[i], sem_j[i]).wait()
                desc = pltpu.async_copy(rows[i], acc.at[dstb[i]],
                                        sem_s[i], add=True)

                @pl.when(k + _NBUF < n)
                def _():
                    desc.wait()
                    pltpu.make_async_copy(
                        src_hbm.at[pl.ds(0, _MK)], srcb[i], sem_i[i]).wait()
                    pltpu.async_copy(
                        dst_hbm.at[pl.ds(ebase + (k + _NBUF) * _MK, _MK)],
                        dstb[i], sem_j[i])
                    pltpu.async_copy(hs_hbm.at[srcb[i]], rows[i], sem_g[i])
        return 0
    lax.fori_loop(0, 27, _group, 0)

    for i in range(_NBUF):
        pltpu.make_async_copy(hs_hbm.at[pl.ds(0, _MK)], rows[i],
                              sem_s[i]).wait()
    plsc.subcore_barrier()

    @pl.when(s < 15)
    def _():
        pltpu.sync_copy(acc.at[pl.ds(s * _RT0, _RT0)],
                        out_hbm.at[c, pl.ds(s * _RT0, _RT0)])

    @pl.when(s == 15)
    def _():
        pltpu.sync_copy(acc.at[pl.ds(15 * _RT0, _RT15)],
                        out_hbm.at[c, pl.ds(15 * _RT0, _RT15)])


_SC_C = 64


@functools.cache
def _segmax_kernel_fn():
    return pl.kernel(
        _segmax_body,
        out_type=jax.ShapeDtypeStruct((G, D), _f32),
        mesh=_mesh(),
        scratch_types=[
            pltpu.VMEM((_SC_C + 8, D), _f32),
            pltpu.VMEM((256,), _f32),
            pltpu.VMEM((16,), _i32),
            pltpu.VMEM((8, D), _f32),
        ],
    )


def _segmax_body(r_hbm, starts_hbm, st_hbm, out_hbm, rowbuf, stv, sv, obuf):
    c = lax.axis_index("c")
    s = lax.axis_index("s")
    wid = c * NS + s
    g0 = wid * 8

    pltpu.sync_copy(starts_hbm.at[pl.ds(g0, 16)], sv)
    pltpu.sync_copy(st_hbm, stv)
    svv = sv[pl.ds(0, 16)]

    ninf = jnp.full((16,), -jnp.inf, _f32)
    for k in range(8):
        gs = svv[k]
        ge = svv[k + 1]
        nch = lax.div(ge - gs + (_SC_C - 1), _SC_C)

        def _chunk(t, carry):
            base = gs + t * _SC_C
            nrows = jnp.minimum(_SC_C, ge - base)
            ab = jnp.minimum(lax.div(base, 8) * 8, N - (_SC_C + 8))
            sh = base - ab
            pltpu.sync_copy(r_hbm.at[pl.ds(ab, _SC_C + 8)], rowbuf)

            def _row(i, acc8):
                return tuple(
                    jnp.maximum(acc8[cc], rowbuf[sh + i, pl.ds(cc * 16, 16)])
                    for cc in range(8))
            return lax.fori_loop(0, nrows, _row, carry)
        acc8 = lax.fori_loop(0, nch, _chunk, (ninf,) * 8)

        for cc in range(8):
            s2 = stv[pl.ds(cc * 16, 16)]
            t2 = stv[pl.ds(D + cc * 16, 16)]
            obuf[k, pl.ds(cc * 16, 16)] = acc8[cc] * s2 + t2
    pltpu.sync_copy(obuf, out_hbm.at[pl.ds(g0, 8)])


def _k1_body(x_ref, deg_ref, w_ref, hs_ref, dinv_ref):
    dinv = lax.rsqrt(deg_ref[...] + 1.0)
    h = jnp.dot(x_ref[...], w_ref[...], preferred_element_type=_f32)
    hs_ref[...] = h * dinv
    dinv_ref[...] = dinv


def _k1(x, degcol, W):
    return pl.pallas_call(
        _k1_body,
        grid=(NBLK,),
        in_specs=[
            pl.BlockSpec((BLK, D), lambda i: (i, 0)),
            pl.BlockSpec((BLK, 1), lambda i: (i, 0)),
            pl.BlockSpec((D, D), lambda i: (0, 0)),
        ],
        out_specs=[
            pl.BlockSpec((BLK, D), lambda i: (i, 0)),
            pl.BlockSpec((BLK, 1), lambda i: (i, 0)),
        ],
        out_shape=[
            jax.ShapeDtypeStruct((N, D), _f32),
            jax.ShapeDtypeStruct((N, 1), _f32),
        ],
    )(x, degcol, W)


def _k23_body(p0_ref, p1_ref, dinv_ref, b_ref, g_ref, be_ref, w_ref,
              hs2_ref, rbuf, ssum, ssq, stscr):
    t = pl.program_id(0)
    i = pl.program_id(1)

    @pl.when(t == 0)
    def _():
        v = (dinv_ref[...] * (p0_ref[...] + p1_ref[...])
             + b_ref[...])
        r = jnp.maximum(v, 0.0)
        rbuf[pl.ds(i * BLK, BLK), :] = r
        cs = jnp.sum(r, axis=0, keepdims=True)
        cq = jnp.sum(r * r, axis=0, keepdims=True)

        @pl.when(i == 0)
        def _():
            ssum[...] = cs
            ssq[...] = cq

        @pl.when(i > 0)
        def _():
            ssum[...] += cs
            ssq[...] += cq

        @pl.when(i == NBLK - 1)
        def _():
            mean = ssum[...] / N
            var = ssq[...] / N - mean * mean
            sc = g_ref[...] * lax.rsqrt(var + EPS)
            stscr[...] = jnp.concatenate(
                [sc, be_ref[...] - mean * sc], axis=0)

    @pl.when(t == 1)
    def _():
        sc = stscr[0:1, :]
        sh = stscr[1:2, :]
        x2 = rbuf[pl.ds(i * BLK, BLK), :] * sc + sh
        h2 = jnp.dot(x2, w_ref[...], preferred_element_type=_f32)
        hs2_ref[...] = h2 * dinv_ref[...]


def _k23(p0, p1, dinvcol, b, g, be, W2):
    blk_p0 = pl.BlockSpec((BLK, D), lambda t, i: (i * (1 - t), 0))
    return pl.pallas_call(
        _k23_body,
        grid=(2, NBLK),
        in_specs=[
            blk_p0,
            blk_p0,
            pl.BlockSpec((BLK, 1), lambda t, i: (i, 0)),
            pl.BlockSpec((1, D), lambda t, i: (0, 0)),
            pl.BlockSpec((1, D), lambda t, i: (0, 0)),
            pl.BlockSpec((1, D), lambda t, i: (0, 0)),
            pl.BlockSpec((D, D), lambda t, i: (0, 0)),
        ],
        out_specs=pl.BlockSpec((BLK, D), lambda t, i: (i * t, 0)),
        out_shape=jax.ShapeDtypeStruct((N, D), _f32),
        scratch_shapes=[
            pltpu.VMEM((N, D), _f32),
            pltpu.VMEM((1, D), _f32),
            pltpu.VMEM((1, D), _f32),
            pltpu.VMEM((2, D), _f32),
        ],
    )(p0, p1, dinvcol, b, g, be, W2)


def _k2_body(p0_ref, p1_ref, dinv_ref, b_ref, g_ref, be_ref,
             r_ref, st_ref, ssum, ssq):
    i = pl.program_id(0)
    v = dinv_ref[...] * (p0_ref[...] + p1_ref[...]) + b_ref[...]
    r = jnp.maximum(v, 0.0)
    r_ref[...] = r
    cs = jnp.sum(r, axis=0, keepdims=True)
    cq = jnp.sum(r * r, axis=0, keepdims=True)

    @pl.when(i == 0)
    def _():
        ssum[...] = cs
        ssq[...] = cq

    @pl.when(i > 0)
    def _():
        ssum[...] += cs
        ssq[...] += cq

    @pl.when(i == NBLK - 1)
    def _():
        mean = ssum[...] / N
        var = ssq[...] / N - mean * mean
        sc = g_ref[...] * lax.rsqrt(var + EPS)
        st_ref[...] = jnp.concatenate([sc, be_ref[...] - mean * sc], axis=0)


def _k2(p0, p1, dinvcol, b, g, be):
    return pl.pallas_call(
        _k2_body,
        grid=(NBLK,),
        in_specs=[
            pl.BlockSpec((BLK, D), lambda i: (i, 0)),
            pl.BlockSpec((BLK, D), lambda i: (i, 0)),
            pl.BlockSpec((BLK, 1), lambda i: (i, 0)),
            pl.BlockSpec((1, D), lambda i: (0, 0)),
            pl.BlockSpec((1, D), lambda i: (0, 0)),
            pl.BlockSpec((1, D), lambda i: (0, 0)),
        ],
        out_specs=[
            pl.BlockSpec((BLK, D), lambda i: (i, 0)),
            pl.BlockSpec((2, D), lambda i: (0, 0)),
        ],
        out_shape=[
            jax.ShapeDtypeStruct((N, D), _f32),
            jax.ShapeDtypeStruct((2, D), _f32),
        ],
        scratch_shapes=[
            pltpu.VMEM((1, D), _f32),
            pltpu.VMEM((1, D), _f32),
        ],
    )(p0, p1, dinvcol, b, g, be)


def kernel(input_feature, input_adj, ibatch, W1, b1, g1, be1, W2, b2, g2, be2):
    src1d = input_adj[0]
    dst2d = input_adj[1].reshape(ER, 128)

    deg, starts = _deg_kernel_fn()(dst2d, ibatch)
    hs1, dinvcol = _k1(input_feature, deg[:, None], W1)

    dst1d = input_adj[1]
    parts1 = _msg_kernel_fn()(hs1, src1d, dst1d)
    hs2 = _k23(parts1[0], parts1[1], dinvcol,
               b1[None, :], g1[None, :], be1[None, :], W2)

    parts2 = _msg_kernel_fn()(hs2, src1d, dst1d)
    r2, st2 = _k2(parts2[0], parts2[1], dinvcol,
                  b2[None, :], g2[None, :], be2[None, :])

    out = _segmax_kernel_fn()(r2, starts, st2.reshape(2 * D))
    return out
